# baseline (speedup 1.0000x reference)
import functools

import jax
import jax.numpy as jnp
from jax import lax
from jax.experimental import pallas as pl
from jax.experimental.pallas import tpu as pltpu
from jax.experimental.pallas import tpu_sc as plsc

_NC = 2
_NS = 16
_NW = _NC * _NS
_L = 16

_MESH = plsc.VectorSubcoreMesh(core_axis_name="c", subcore_axis_name="s",
                               num_cores=_NC, num_subcores=_NS)

N = 10000
E = 160000
D = 128
P = 20000
NF = 1200
NR = 30

_W = 3 * D + 32
_WC = D + 8

_NEG = -1e30


def _erf(x):
    s = jnp.sign(x)
    a = jnp.abs(x)
    t = 1.0 / (1.0 + 0.3275911 * a)
    poly = t * (0.254829592 + t * (-0.284496736 + t * (1.421413741 +
           t * (-1.453152027 + t * 1.061405429))))
    return s * (1.0 - poly * jnp.exp(-a * a))


def _gelu(x):
    return 0.5 * x * (1.0 + _erf(x * 0.7071067811865476))


def _tables_body(emb, kwn, kbn, qwn, qbn, vwn, vbn, art, mrt, mri,
                 kwe, kbe, aro, vwe, vbe, mro, awe, abe, rwt2,
                 scal,
                 q_tab, mcs, mce):
    x = emb[...]
    a_e = scal[0, 0]
    ct = scal[0, 1]
    co = scal[0, 2]
    q_tab[...] = x @ qwn[...] + qbn[...]
    krt = (x @ (kwn[...] @ art[...]) + kbn[...] @ art[...]) * ct
    vrt = x @ (vwn[...] @ mrt[...]) + vbn[...] @ mrt[...]
    kro = (x @ (kwe[...] @ aro[...]) + kbe[...] @ aro[...]) * co
    vro = x @ (vwe[...] @ mro[...]) + vbe[...] @ mro[...]
    vrin = x @ (vwn[...] @ mri[...]) + vbn[...] @ mri[...]
    hidA = _gelu(vrin) @ awe[...] + abe[...]
    hidA2 = a_e * hidA
    embB = (1.0 - a_e) * x
    mcs[...] = jnp.concatenate([krt, hidA2, vrt, hidA2 @ rwt2[...]], axis=1)
    mce[...] = jnp.concatenate([kro, embB, vro, embB @ rwt2[...]], axis=1)


def _make_tables(emb, prm, a_e, rwt2):
    bp = 2000
    grid = (P // bp,)
    scal = jnp.stack([a_e,
                      prm['p_rel_true'] / jnp.sqrt(jnp.float32(D)),
                      prm['p_rel_out'] / jnp.sqrt(jnp.float32(D))]).reshape(1, 3)

    def rep(shape):
        return pl.BlockSpec(shape, lambda i: (0,) * len(shape))

    dd = rep((D, D))
    db = rep((1, D))
    in_specs = [pl.BlockSpec((bp, D), lambda i: (i, 0)),
                dd, db, dd, db, dd, db, dd, dd, dd,
                dd, db, dd, dd, db, dd, dd, db, rep((D, 32)),
                rep((1, 3))]
    out_specs = [pl.BlockSpec((bp, D), lambda i: (i, 0)),
                 pl.BlockSpec((bp, _W), lambda i: (i, 0)),
                 pl.BlockSpec((bp, _W), lambda i: (i, 0))]
    out_shape = [jax.ShapeDtypeStruct((P, D), jnp.float32),
                 jax.ShapeDtypeStruct((P, _W), jnp.float32),
                 jax.ShapeDtypeStruct((P, _W), jnp.float32)]
    args = (emb,
            prm['k_w_node'], prm['k_b_node'].reshape(1, D),
            prm['q_w_node'], prm['q_b_node'].reshape(1, D),
            prm['v_w_node'], prm['v_b_node'].reshape(1, D),
            prm['a_rel_true'], prm['m_rel_true'], prm['m_rel_in'],
            prm['k_w_edge'], prm['k_b_edge'].reshape(1, D),
            prm['a_rel_out'],
            prm['v_w_edge'], prm['v_b_edge'].reshape(1, D),
            prm['m_rel_out'],
            prm['a_w_edge'], prm['a_b_edge'].reshape(1, D),
            rwt2, scal)
    return pl.pallas_call(
        _tables_body, grid=grid, in_specs=in_specs, out_specs=out_specs,
        out_shape=out_shape)(*args)


def _sc_big_gather(mcs, mce, q_tab, node_x, edge_src, edge_dst, edge_x,
                   chunk=64):
    nch = E // chunk
    assert E % chunk == 0 and chunk % _L == 0 and chunk <= 128

    scratch = [pltpu.VMEM((N,), jnp.int32),
               pltpu.VMEM((chunk,), jnp.int32),
               pltpu.VMEM((chunk,), jnp.int32),
               pltpu.VMEM((chunk,), jnp.int32),
               pltpu.VMEM((chunk,), jnp.int32),
               pltpu.VMEM((chunk,), jnp.int32),
               pltpu.VMEM((chunk, _W), jnp.float32),
               pltpu.VMEM((chunk, _W), jnp.float32),
               pltpu.VMEM((chunk, D), jnp.float32),
               pltpu.SemaphoreType.DMA]
    out_type = [jax.ShapeDtypeStruct((E, _W), jnp.float32),
                jax.ShapeDtypeStruct((E, _W), jnp.float32),
                jax.ShapeDtypeStruct((E, D), jnp.float32)]

    @functools.partial(pl.kernel, out_type=out_type, mesh=_MESH,
                       scratch_types=scratch,
                       compiler_params=pltpu.CompilerParams(
                           needs_layout_passes=False,
                           use_tc_tiling_on_sc=False))
    def k(mcs_h, mce_h, q_h, nx_h, es_h, ed_h, ex_h,
          ocs_h, oce_h, oq_h,
          nx_v, es_v, ed_v, ex_v, cs_v, cd_v, bcs_v, bce_v, bq_v, sem):
        wid = lax.axis_index("s") * _NC + lax.axis_index("c")
        pltpu.sync_copy(nx_h, nx_v)
        nloc = (nch - wid + _NW - 1) // _NW

        def body(j, _):
            base = (wid + j * _NW) * chunk
            pltpu.sync_copy(es_h.at[pl.ds(base, chunk)], es_v)
            pltpu.sync_copy(ed_h.at[pl.ds(base, chunk)], ed_v)
            pltpu.sync_copy(ex_h.at[pl.ds(base, chunk)], ex_v)
            for g in range(chunk // _L):
                sl = pl.ds(g * _L, _L)
                cs_v[sl] = plsc.load_gather(nx_v, [es_v[sl]])
                cd_v[sl] = plsc.load_gather(nx_v, [ed_v[sl]])
            h1 = pltpu.async_copy(mcs_h.at[cs_v], bcs_v, sem)
            h2 = pltpu.async_copy(mce_h.at[ex_v], bce_v, sem)
            h3 = pltpu.async_copy(q_h.at[cd_v], bq_v, sem)
            h1.wait()
            h2.wait()
            h3.wait()
            pltpu.sync_copy(bcs_v, ocs_h.at[pl.ds(base, chunk)])
            pltpu.sync_copy(bce_v, oce_h.at[pl.ds(base, chunk)])
            pltpu.sync_copy(bq_v, oq_h.at[pl.ds(base, chunk)])
            return 0

        lax.fori_loop(0, nloc, body, 0)

    return k(mcs, mce, q_tab, node_x, edge_src, edge_dst, edge_x)


def _sc_gather_rows(tables, idxs, chunk=128):
    np_ = len(tables)
    etot = idxs[0].shape[0]
    assert etot % chunk == 0 and chunk % _L == 0 and chunk <= 128
    nch = etot // chunk

    scratch = ([pltpu.VMEM((chunk,), jnp.int32) for _ in range(np_)] +
               [pltpu.VMEM((chunk, t.shape[1]), t.dtype) for t in tables] +
               [pltpu.SemaphoreType.DMA])
    out_type = [jax.ShapeDtypeStruct((etot, t.shape[1]), t.dtype)
                for t in tables]

    @functools.partial(pl.kernel, out_type=out_type, mesh=_MESH,
                       scratch_types=scratch,
                       compiler_params=pltpu.CompilerParams(
                           needs_layout_passes=False,
                           use_tc_tiling_on_sc=False))
    def k(*refs):
        tab_h = refs[:np_]
        idx_h = refs[np_:2 * np_]
        out_h = refs[2 * np_:3 * np_]
        idx_v = refs[3 * np_:4 * np_]
        rows_v = refs[4 * np_:5 * np_]
        sem = refs[5 * np_]
        wid = lax.axis_index("s") * _NC + lax.axis_index("c")
        nloc = (nch - wid + _NW - 1) // _NW

        def body(j, _):
            base = (wid + j * _NW) * chunk
            for p in range(np_):
                pltpu.sync_copy(idx_h[p].at[pl.ds(base, chunk)], idx_v[p])
            handles = [pltpu.async_copy(tab_h[p].at[idx_v[p]], rows_v[p], sem)
                       for p in range(np_)]
            for h in handles:
                h.wait()
            for p in range(np_):
                pltpu.sync_copy(rows_v[p], out_h[p].at[pl.ds(base, chunk)])
            return 0

        lax.fori_loop(0, nloc, body, 0)

    return k(*tables, *idxs)


def _sc_gather_scalars(tables, idxs, chunk=640):
    np_ = len(tables)
    etot = idxs[0].shape[0]
    assert etot % chunk == 0 and chunk % _L == 0
    nch = etot // chunk

    scratch = ([pltpu.VMEM(t.shape, t.dtype) for t in tables] +
               [pltpu.VMEM((chunk,), jnp.int32) for _ in range(np_)] +
               [pltpu.VMEM((chunk,), t.dtype) for t in tables])
    out_type = [jax.ShapeDtypeStruct((etot,), t.dtype) for t in tables]

    @functools.partial(pl.kernel, out_type=out_type, mesh=_MESH,
                       scratch_types=scratch,
                       compiler_params=pltpu.CompilerParams(
                           needs_layout_passes=False,
                           use_tc_tiling_on_sc=False))
    def k(*refs):
        tab_h = refs[:np_]
        idx_h = refs[np_:2 * np_]
        out_h = refs[2 * np_:3 * np_]
        tab_v = refs[3 * np_:4 * np_]
        idx_v = refs[4 * np_:5 * np_]
        val_v = refs[5 * np_:6 * np_]
        wid = lax.axis_index("s") * _NC + lax.axis_index("c")
        for p in range(np_):
            pltpu.sync_copy(tab_h[p], tab_v[p])
        nloc = (nch - wid + _NW - 1) // _NW

        def body(j, _):
            base = (wid + j * _NW) * chunk
            for p in range(np_):
                pltpu.sync_copy(idx_h[p].at[pl.ds(base, chunk)], idx_v[p])
            for p in range(np_):
                for g in range(chunk // _L):
                    iv = idx_v[p][pl.ds(g * _L, _L)]
                    val_v[p][pl.ds(g * _L, _L)] = plsc.load_gather(tab_v[p], [iv])
                pltpu.sync_copy(val_v[p], out_h[p].at[pl.ds(base, chunk)])
            return 0

        lax.fori_loop(0, nloc, body, 0)

    return k(*tables, *idxs)


def _sc_scatter_scalars(vals, dst, nseg, chunk=640):
    nv = len(vals)
    etot = dst.shape[0]
    assert etot % chunk == 0 and chunk % _L == 0 and nseg % _L == 0
    nch = etot // chunk

    scratch = ([pltpu.VMEM((nseg,), jnp.float32) for _ in range(nv)] +
               [pltpu.VMEM((chunk,), jnp.int32)] +
               [pltpu.VMEM((chunk,), jnp.float32) for _ in range(nv)])
    out_type = jax.ShapeDtypeStruct((_NW, nv, nseg), jnp.float32)

    @functools.partial(pl.kernel, out_type=out_type, mesh=_MESH,
                       scratch_types=scratch,
                       compiler_params=pltpu.CompilerParams(
                           needs_layout_passes=False,
                           use_tc_tiling_on_sc=False))
    def k(*refs):
        val_h = refs[:nv]
        dst_h = refs[nv]
        out_h = refs[nv + 1]
        acc_v = refs[nv + 2:nv + 2 + nv]
        dst_v = refs[nv + 2 + nv]
        val_v = refs[nv + 3 + nv:nv + 3 + 2 * nv]
        wid = lax.axis_index("s") * _NC + lax.axis_index("c")

        def zero(i, _):
            for a in acc_v:
                a[pl.ds(i * _L, _L)] = jnp.zeros((_L,), jnp.float32)
            return 0

        lax.fori_loop(0, nseg // _L, zero, 0)
        nloc = (nch - wid + _NW - 1) // _NW

        def body(j, _):
            base = (wid + j * _NW) * chunk
            pltpu.sync_copy(dst_h.at[pl.ds(base, chunk)], dst_v)
            for p in range(nv):
                pltpu.sync_copy(val_h[p].at[pl.ds(base, chunk)], val_v[p])
            for g in range(chunk // _L):
                dv = dst_v[pl.ds(g * _L, _L)]
                for p in range(nv):
                    plsc.addupdate_scatter(acc_v[p], [dv],
                                           val_v[p][pl.ds(g * _L, _L)])
            return 0

        lax.fori_loop(0, nloc, body, 0)
        for p in range(nv):
            pltpu.sync_copy(acc_v[p], out_h.at[wid, p])

    return k(*vals, dst)


def _sc_scatter_rows(rows, dst, zeros, nseg, chunk=128):
    _, etot, w = rows.shape
    assert etot % chunk == 0 and chunk <= 128 and nseg % _NS == 0
    assert w == _WC
    nch = etot // chunk
    rows_per_tile = nseg // _NS

    scratch = [pltpu.VMEM_SHARED((nseg, _WC), jnp.float32),
               pltpu.VMEM((chunk,), jnp.int32),
               pltpu.VMEM((chunk, _WC), jnp.float32)]
    out_type = jax.ShapeDtypeStruct((_NC, nseg, _WC), jnp.float32)

    @functools.partial(pl.kernel, out_type=out_type, mesh=_MESH,
                       scratch_types=scratch,
                       compiler_params=pltpu.CompilerParams(
                           needs_layout_passes=False,
                           use_tc_tiling_on_sc=False))
    def k(rows_h, dst_h, zeros_h, out_h, acc_s, dst_v, rows_v):
        cid = lax.axis_index("c")
        sid = lax.axis_index("s")
        row0 = sid * rows_per_tile
        pltpu.sync_copy(zeros_h.at[pl.ds(row0, rows_per_tile)],
                        acc_s.at[pl.ds(row0, rows_per_tile)])
        plsc.subcore_barrier()
        nloc = (nch - sid + _NS - 1) // _NS

        def body(j, _):
            base = (sid + j * _NS) * chunk
            pltpu.sync_copy(dst_h.at[pl.ds(base, chunk)], dst_v)
            pltpu.sync_copy(rows_h.at[cid, pl.ds(base, chunk)], rows_v)
            pltpu.sync_copy(rows_v, acc_s.at[dst_v], add=True)
            return 0

        lax.fori_loop(0, nloc, body, 0)
        plsc.subcore_barrier()
        pltpu.sync_copy(acc_s.at[pl.ds(row0, rows_per_tile)],
                        out_h.at[cid, pl.ds(row0, rows_per_tile)])

    return k(rows, dst, zeros)


def _score_body(qg, krtg, vrtg, krog, vrog, wrows):
    q = qg[...]
    et = jnp.exp(jnp.sum(krtg[...] * q, axis=1, keepdims=True))
    eo = jnp.exp(jnp.sum(krog[...] * q, axis=1, keepdims=True))
    ones = jnp.ones_like(et)
    pad = jnp.zeros((et.shape[0], 5), jnp.float32)
    blk_a = jnp.concatenate([et * vrtg[...], et, eo, ones, pad], axis=1)
    blk_b = jnp.concatenate([eo * vrog[...], et, eo, ones, pad], axis=1)
    wrows[...] = jnp.stack([blk_a, blk_b])


def _score_stage(qg, ocs, oce):
    be = 2000
    grid = (E // be,)
    in_specs = [pl.BlockSpec((be, D), lambda i: (i, 0)),
                pl.BlockSpec((be, D), lambda i: (i, 0)),
                pl.BlockSpec((be, D), lambda i: (i, 2)),
                pl.BlockSpec((be, D), lambda i: (i, 0)),
                pl.BlockSpec((be, D), lambda i: (i, 2))]
    return pl.pallas_call(
        _score_body, grid=grid, in_specs=in_specs,
        out_specs=pl.BlockSpec((2, be, _WC), lambda i: (0, i, 0)),
        out_shape=jax.ShapeDtypeStruct((2, E, _WC), jnp.float32),
    )(qg, ocs, ocs, oce, oce)


def _prep_body(ha2g, rag, ebg, rbg, eproj, rsum, geacc):
    ra = rag[...]
    rb = rbg[...]
    eproj[...] = ra[:, 30:31] + rb[:, 30:31]
    rsum[...] = ra[:, :32] + rb[:, :32]
    z = ha2g[...] + ebg[...]
    ez = jnp.exp(z)
    nm = jnp.sum(ez * z, axis=0, keepdims=True)
    dn = jnp.sum(ez, axis=0, keepdims=True)
    blk = jnp.concatenate([nm, dn], axis=0).reshape(1, 2, D)

    @pl.when(pl.program_id(0) == 0)
    def _():
        geacc[...] = blk

    @pl.when(pl.program_id(0) != 0)
    def _():
        geacc[...] += blk


def _prep_stage(ocs, oce):
    be = 2000
    grid = (E // be,)
    in_specs = [pl.BlockSpec((be, D), lambda i: (i, 1)),
                pl.BlockSpec((be, D), lambda i: (i, 3)),
                pl.BlockSpec((be, D), lambda i: (i, 1)),
                pl.BlockSpec((be, D), lambda i: (i, 3))]
    out_specs = [pl.BlockSpec((be, 1), lambda i: (i, 0)),
                 pl.BlockSpec((be, 32), lambda i: (i, 0)),
                 pl.BlockSpec((1, 2, D), lambda i: (0, 0, 0))]
    out_shape = [jax.ShapeDtypeStruct((E, 1), jnp.float32),
                 jax.ShapeDtypeStruct((E, 32), jnp.float32),
                 jax.ShapeDtypeStruct((1, 2, D), jnp.float32)]
    return pl.pallas_call(
        _prep_body, grid=grid, in_specs=in_specs, out_specs=out_specs,
        out_shape=out_shape)(ocs, ocs, oce, oce)


def _node_body(aggp, x_node, awn, abn, glw, grw, scal, xl, xr, cnt_o, gnp):
    a = aggp[...]
    num_t = a[0][:, :D]
    num_o = a[1][:, :D]
    den_t = a[0][:, D:D + 1]
    den_o = a[1][:, D + 1:D + 2]
    cnt_o[...] = a[0][:, D + 2:D + 3]
    agg = (num_t / (den_t + 1e-16) +
           num_o / (den_o + 1e-16))
    a_n = scal[0, 0]
    o = _gelu(agg) @ awn[...] + abn[...]
    hid = a_n * o + (1.0 - a_n) * x_node[...]
    xl[...] = hid @ glw[...] + scal[0, 1]
    xr[...] = hid @ grw[...] + scal[0, 2]
    mb = jnp.max(hid, axis=0, keepdims=True)
    @pl.when(pl.program_id(0) == 0)
    def _():
        ez = jnp.exp(hid - mb)
        gnp[...] = jnp.concatenate(
            [mb, jnp.sum(ez * hid, axis=0, keepdims=True),
             jnp.sum(ez, axis=0, keepdims=True)], axis=0)

    @pl.when(pl.program_id(0) != 0)
    def _():
        g = gnp[...]
        m_old = g[0:1]
        m_new = jnp.maximum(m_old, mb)
        ez = jnp.exp(hid - m_new)
        sc = jnp.exp(m_old - m_new)
        gnp[...] = jnp.concatenate(
            [m_new,
             g[1:2] * sc + jnp.sum(ez * hid, axis=0, keepdims=True),
             g[2:3] * sc + jnp.sum(ez, axis=0, keepdims=True)], axis=0)


def _node_stage(agg_partials, x_node, prm):
    bn = 2000
    scal = jnp.stack([jax.nn.sigmoid(prm['skip_node']),
                      prm['gat_l_b'][0], prm['gat_r_b'][0]]).reshape(1, 3)

    def rep(shape):
        return pl.BlockSpec(shape, lambda i: (0,) * len(shape))

    return pl.pallas_call(
        _node_body, grid=(N // bn,),
        in_specs=[pl.BlockSpec((2, bn, _WC), lambda i: (0, i, 0)),
                  pl.BlockSpec((bn, D), lambda i: (i, 0)),
                  rep((D, D)), rep((1, D)), rep((D, 1)), rep((D, 1)),
                  rep((1, 3))],
        out_specs=[pl.BlockSpec((bn, 1), lambda i: (i, 0)),
                   pl.BlockSpec((bn, 1), lambda i: (i, 0)),
                   pl.BlockSpec((bn, 1), lambda i: (i, 0)),
                   pl.BlockSpec((3, D), lambda i: (0, 0))],
        out_shape=[jax.ShapeDtypeStruct((N, 1), jnp.float32),
                   jax.ShapeDtypeStruct((N, 1), jnp.float32),
                   jax.ShapeDtypeStruct((N, 1), jnp.float32),
                   jax.ShapeDtypeStruct((3, D), jnp.float32)],
    )(agg_partials, x_node, prm['a_w_node'], prm['a_b_node'].reshape(1, D),
      prm['gat_l_w'], prm['gat_r_w'], scal)


def _gat_edge_body(xls, xrd, ep, scal, es, esx):
    z = xls[...] + xrd[...] + ep[...]
    s = jnp.maximum(z, 0.2 * z) * scal[0, 0]
    e = jnp.exp(s)
    es[...] = e
    esx[...] = e * xls[...]


def _gat_edge_stage(xls, xrd, eproj, prm):
    be = 2000
    scal = prm['gat_att'].reshape(1, 1)
    return pl.pallas_call(
        _gat_edge_body, grid=(E // be,),
        in_specs=[pl.BlockSpec((be, 1), lambda i: (i, 0))] * 3 +
                 [pl.BlockSpec((1, 1), lambda i: (0, 0))],
        out_specs=[pl.BlockSpec((be, 1), lambda i: (i, 0))] * 2,
        out_shape=[jax.ShapeDtypeStruct((E, 1), jnp.float32)] * 2,
    )(xls.reshape(E, 1), xrd.reshape(E, 1), eproj, scal)


def _rootframe_body(gatp, cntc, xl, xr, gnp, gep, fw, fb, rwb, rb, scal,
                    root_preds, amax, frame, const32):
    g = jnp.sum(gatp[...], axis=0)
    cnt = cntc[...].reshape(1, N)
    att = scal[0, 0]
    bias = scal[0, 1]
    xlr = xl[...].reshape(1, N)
    xrr = xr[...].reshape(1, N)
    loop_eproj = g[0:1] / jnp.maximum(cnt, 1.0)
    z = xlr + xrr + loop_eproj
    s_self = jnp.maximum(z, 0.2 * z) * att
    es = jnp.exp(s_self)
    den = g[1:2] + es
    num = g[2:3] + es * xlr
    root = num / (den + 1e-16) + bias
    m = jnp.max(root, axis=1, keepdims=True)
    e = jnp.exp(root - m)
    lse = jnp.log(jnp.sum(e, axis=1, keepdims=True))
    root_preds[...] = root - m - lse
    idx = lax.broadcasted_iota(jnp.int32, (1, N), 1)
    amax[...] = jnp.min(jnp.where(root == m, idx, N), axis=1, keepdims=True)
    num2 = jnp.sum(gep[...], axis=0)
    ge = num2[0:1] / num2[1:2]
    gg = gnp[...]
    gn = gg[1:2] / gg[2:3]
    grep = jnp.concatenate([gn, ge], axis=1)
    f = grep @ fw[...] + fb[...]
    mf = jnp.max(f, axis=1, keepdims=True)
    lsef = jnp.log(jnp.sum(jnp.exp(f - mf), axis=1, keepdims=True))
    frame[...] = f - mf - lsef
    const32[...] = gn @ rwb[...] + rb[...]


def _rootframe_stage(gat_partials, cntc, xl, xr, gnp, ge_partials, prm,
                     rwb_pad, rb_pad):
    scal = jnp.stack([prm['gat_att'][0], prm['gat_bias'][0]]).reshape(1, 2)
    return pl.pallas_call(
        _rootframe_body,
        out_shape=[jax.ShapeDtypeStruct((1, N), jnp.float32),
                   jax.ShapeDtypeStruct((1, 1), jnp.int32),
                   jax.ShapeDtypeStruct((1, NF), jnp.float32),
                   jax.ShapeDtypeStruct((1, 32), jnp.float32)],
    )(gat_partials, cntc, xl, xr, gnp, ge_partials,
      prm['frame_w'], prm['frame_b'].reshape(1, NF), rwb_pad, rb_pad, scal)


def _role_body(rsum, src, amax, const32, out):
    r = rsum[...] + const32[...]
    keep = src[...] == amax[0, 0]
    r = jnp.where(keep, r, 0.0)
    lane = lax.broadcasted_iota(jnp.int32, r.shape, 1)
    valid = lane < NR
    rm = jnp.where(valid, r, _NEG)
    m = jnp.max(rm, axis=1, keepdims=True)
    e = jnp.where(valid, jnp.exp(r - m), 0.0)
    lse = jnp.log(jnp.sum(e, axis=1, keepdims=True))
    out[...] = r - m - lse


def _role_stage(rsum, edge_src, amax, const32):
    be = 2000
    grid = (E // be,)
    return pl.pallas_call(
        _role_body, grid=grid,
        in_specs=[pl.BlockSpec((be, 32), lambda i: (i, 0)),
                  pl.BlockSpec((be, 1), lambda i: (i, 0)),
                  pl.BlockSpec((1, 1), lambda i: (0, 0)),
                  pl.BlockSpec((1, 32), lambda i: (0, 0))],
        out_specs=pl.BlockSpec((be, 32), lambda i: (i, 0)),
        out_shape=jax.ShapeDtypeStruct((E, 32), jnp.float32),
    )(rsum, edge_src.reshape(E, 1), amax, const32)


def kernel(node_x, edge_x, edge_src, edge_dst, params):
    prm = params
    emb = prm['pred_emb']
    a_e = jax.nn.sigmoid(prm['skip_edge'])
    rwt2 = jnp.concatenate(
        [prm['role_w'][:D], prm['gat_e_w'],
         jnp.zeros((D, 32 - NR - 1), jnp.float32)], axis=1)
    rwb_pad = jnp.pad(prm['role_w'][D:], ((0, 0), (0, 32 - NR)))
    rb_pad = jnp.pad(prm['role_b'], (0, 32 - NR)).reshape(1, 32)

    q_tab, mcs, mce = _make_tables(emb, prm, a_e, rwt2)

    (x_node,) = _sc_gather_rows([emb], [node_x], chunk=80)

    ocs, oce, qg = _sc_big_gather(mcs, mce, q_tab, node_x,
                                  edge_src, edge_dst, edge_x)

    wrows = _score_stage(qg, ocs, oce)
    eproj, rsum, ge_partials = _prep_stage(ocs, oce)

    zeros_nw = jnp.zeros((N, _WC), jnp.float32)
    agg_partials = _sc_scatter_rows(wrows, edge_dst, zeros_nw, N)

    xl, xr, cntc, gnp = _node_stage(agg_partials, x_node, prm)

    xls, xrd = _sc_gather_scalars([xl.reshape(N), xr.reshape(N)],
                                  [edge_src, edge_dst])
    es2, esx2 = _gat_edge_stage(xls, xrd, eproj, prm)
    gat_partials = _sc_scatter_scalars(
        [eproj.reshape(E), es2.reshape(E), esx2.reshape(E)], edge_dst, N)

    root_preds2, amax, frame2, const32 = _rootframe_stage(
        gat_partials, cntc, xl, xr, gnp, ge_partials, prm, rwb_pad, rb_pad)

    role32 = _role_stage(rsum, edge_src, amax, const32)

    root_preds = root_preds2.reshape(N)
    frame_preds = frame2.reshape(NF)
    role_preds = role32[:, :NR]
    return ((root_preds, frame_preds), role_preds)


# speedup vs baseline: 1.1918x; 1.1918x over previous
import functools

import jax
import jax.numpy as jnp
from jax import lax
from jax.experimental import pallas as pl
from jax.experimental.pallas import tpu as pltpu
from jax.experimental.pallas import tpu_sc as plsc

_NC = 2
_NS = 16
_NW = _NC * _NS
_L = 16

_MESH = plsc.VectorSubcoreMesh(core_axis_name="c", subcore_axis_name="s",
                               num_cores=_NC, num_subcores=_NS)

N = 10000
E = 160000
D = 128
P = 20000
NF = 1200
NR = 30

_W = 3 * D + 32
_WC = D + 8

_NEG = -1e30


def _erf(x):
    s = jnp.sign(x)
    a = jnp.abs(x)
    t = 1.0 / (1.0 + 0.3275911 * a)
    poly = t * (0.254829592 + t * (-0.284496736 + t * (1.421413741 +
           t * (-1.453152027 + t * 1.061405429))))
    return s * (1.0 - poly * jnp.exp(-a * a))


def _gelu(x):
    return 0.5 * x * (1.0 + _erf(x * 0.7071067811865476))


def _tables_body(emb, kwn, kbn, qwn, qbn, vwn, vbn, art, mrt, mri,
                 kwe, kbe, aro, vwe, vbe, mro, awe, abe, rwt2,
                 scal,
                 q_tab, mcs, mce):
    x = emb[...]
    a_e = scal[0, 0]
    ct = scal[0, 1]
    co = scal[0, 2]
    q_tab[...] = x @ qwn[...] + qbn[...]
    krt = (x @ (kwn[...] @ art[...]) + kbn[...] @ art[...]) * ct
    vrt = x @ (vwn[...] @ mrt[...]) + vbn[...] @ mrt[...]
    kro = (x @ (kwe[...] @ aro[...]) + kbe[...] @ aro[...]) * co
    vro = x @ (vwe[...] @ mro[...]) + vbe[...] @ mro[...]
    vrin = x @ (vwn[...] @ mri[...]) + vbn[...] @ mri[...]
    hidA = _gelu(vrin) @ awe[...] + abe[...]
    hidA2 = a_e * hidA
    embB = (1.0 - a_e) * x
    mcs[...] = jnp.concatenate([krt, hidA2, vrt, hidA2 @ rwt2[...]], axis=1)
    mce[...] = jnp.concatenate([kro, embB, vro, embB @ rwt2[...]], axis=1)


def _make_tables(emb, prm, a_e, rwt2):
    bp = 2000
    grid = (P // bp,)
    scal = jnp.stack([a_e,
                      prm['p_rel_true'] / jnp.sqrt(jnp.float32(D)),
                      prm['p_rel_out'] / jnp.sqrt(jnp.float32(D))]).reshape(1, 3)

    def rep(shape):
        return pl.BlockSpec(shape, lambda i: (0,) * len(shape))

    dd = rep((D, D))
    db = rep((1, D))
    in_specs = [pl.BlockSpec((bp, D), lambda i: (i, 0)),
                dd, db, dd, db, dd, db, dd, dd, dd,
                dd, db, dd, dd, db, dd, dd, db, rep((D, 32)),
                rep((1, 3))]
    out_specs = [pl.BlockSpec((bp, D), lambda i: (i, 0)),
                 pl.BlockSpec((bp, _W), lambda i: (i, 0)),
                 pl.BlockSpec((bp, _W), lambda i: (i, 0))]
    out_shape = [jax.ShapeDtypeStruct((P, D), jnp.float32),
                 jax.ShapeDtypeStruct((P, _W), jnp.float32),
                 jax.ShapeDtypeStruct((P, _W), jnp.float32)]
    args = (emb,
            prm['k_w_node'], prm['k_b_node'].reshape(1, D),
            prm['q_w_node'], prm['q_b_node'].reshape(1, D),
            prm['v_w_node'], prm['v_b_node'].reshape(1, D),
            prm['a_rel_true'], prm['m_rel_true'], prm['m_rel_in'],
            prm['k_w_edge'], prm['k_b_edge'].reshape(1, D),
            prm['a_rel_out'],
            prm['v_w_edge'], prm['v_b_edge'].reshape(1, D),
            prm['m_rel_out'],
            prm['a_w_edge'], prm['a_b_edge'].reshape(1, D),
            rwt2, scal)
    return pl.pallas_call(
        _tables_body, grid=grid, in_specs=in_specs, out_specs=out_specs,
        out_shape=out_shape)(*args)


def _sc_big_gather(mcs, mce, q_tab, node_x, edge_src, edge_dst, edge_x,
                   chunk=64):
    nch = E // chunk
    assert E % chunk == 0 and chunk % _L == 0 and chunk <= 128

    scratch = [pltpu.VMEM((N,), jnp.int32),
               pltpu.VMEM((chunk,), jnp.int32),
               pltpu.VMEM((chunk,), jnp.int32),
               pltpu.VMEM((chunk,), jnp.int32),
               pltpu.VMEM((chunk,), jnp.int32),
               pltpu.VMEM((chunk,), jnp.int32),
               pltpu.VMEM((chunk, _W), jnp.float32),
               pltpu.VMEM((chunk, _W), jnp.float32),
               pltpu.VMEM((chunk, D), jnp.float32),
               pltpu.SemaphoreType.DMA]
    fd = jax.ShapeDtypeStruct((E, D), jnp.float32)
    f32 = jax.ShapeDtypeStruct((E, 32), jnp.float32)
    out_type = [fd, fd, fd, f32, fd, fd, fd, f32, fd]

    @functools.partial(pl.kernel, out_type=out_type, mesh=_MESH,
                       scratch_types=scratch,
                       compiler_params=pltpu.CompilerParams(
                           needs_layout_passes=False,
                           use_tc_tiling_on_sc=False))
    def k(mcs_h, mce_h, q_h, nx_h, es_h, ed_h, ex_h,
          krt_h, ha2_h, vrt_h, ra_h, kro_h, eb_h, vro_h, rb_h, oq_h,
          nx_v, es_v, ed_v, ex_v, cs_v, cd_v, bcs_v, bce_v, bq_v, sem):
        wid = lax.axis_index("s") * _NC + lax.axis_index("c")
        pltpu.sync_copy(nx_h, nx_v)
        nloc = (nch - wid + _NW - 1) // _NW
        rows = pl.ds(0, chunk)

        def body(j, _):
            base = (wid + j * _NW) * chunk
            pltpu.sync_copy(es_h.at[pl.ds(base, chunk)], es_v)
            pltpu.sync_copy(ed_h.at[pl.ds(base, chunk)], ed_v)
            pltpu.sync_copy(ex_h.at[pl.ds(base, chunk)], ex_v)
            for g in range(chunk // _L):
                sl = pl.ds(g * _L, _L)
                cs_v[sl] = plsc.load_gather(nx_v, [es_v[sl]])
                cd_v[sl] = plsc.load_gather(nx_v, [ed_v[sl]])
            h1 = pltpu.async_copy(mcs_h.at[cs_v], bcs_v, sem)
            h2 = pltpu.async_copy(mce_h.at[ex_v], bce_v, sem)
            h3 = pltpu.async_copy(q_h.at[cd_v], bq_v, sem)
            h1.wait()
            h2.wait()
            h3.wait()
            dst = pl.ds(base, chunk)
            pltpu.sync_copy(bcs_v.at[rows, pl.ds(0, D)], krt_h.at[dst])
            pltpu.sync_copy(bcs_v.at[rows, pl.ds(D, D)], ha2_h.at[dst])
            pltpu.sync_copy(bcs_v.at[rows, pl.ds(2 * D, D)], vrt_h.at[dst])
            pltpu.sync_copy(bcs_v.at[rows, pl.ds(3 * D, 32)], ra_h.at[dst])
            pltpu.sync_copy(bce_v.at[rows, pl.ds(0, D)], kro_h.at[dst])
            pltpu.sync_copy(bce_v.at[rows, pl.ds(D, D)], eb_h.at[dst])
            pltpu.sync_copy(bce_v.at[rows, pl.ds(2 * D, D)], vro_h.at[dst])
            pltpu.sync_copy(bce_v.at[rows, pl.ds(3 * D, 32)], rb_h.at[dst])
            pltpu.sync_copy(bq_v, oq_h.at[dst])
            return 0

        lax.fori_loop(0, nloc, body, 0)

    return k(mcs, mce, q_tab, node_x, edge_src, edge_dst, edge_x)


def _sc_gather_rows(tables, idxs, chunk=128):
    np_ = len(tables)
    etot = idxs[0].shape[0]
    assert etot % chunk == 0 and chunk % _L == 0 and chunk <= 128
    nch = etot // chunk

    scratch = ([pltpu.VMEM((chunk,), jnp.int32) for _ in range(np_)] +
               [pltpu.VMEM((chunk, t.shape[1]), t.dtype) for t in tables] +
               [pltpu.SemaphoreType.DMA])
    out_type = [jax.ShapeDtypeStruct((etot, t.shape[1]), t.dtype)
                for t in tables]

    @functools.partial(pl.kernel, out_type=out_type, mesh=_MESH,
                       scratch_types=scratch,
                       compiler_params=pltpu.CompilerParams(
                           needs_layout_passes=False,
                           use_tc_tiling_on_sc=False))
    def k(*refs):
        tab_h = refs[:np_]
        idx_h = refs[np_:2 * np_]
        out_h = refs[2 * np_:3 * np_]
        idx_v = refs[3 * np_:4 * np_]
        rows_v = refs[4 * np_:5 * np_]
        sem = refs[5 * np_]
        wid = lax.axis_index("s") * _NC + lax.axis_index("c")
        nloc = (nch - wid + _NW - 1) // _NW

        def body(j, _):
            base = (wid + j * _NW) * chunk
            for p in range(np_):
                pltpu.sync_copy(idx_h[p].at[pl.ds(base, chunk)], idx_v[p])
            handles = [pltpu.async_copy(tab_h[p].at[idx_v[p]], rows_v[p], sem)
                       for p in range(np_)]
            for h in handles:
                h.wait()
            for p in range(np_):
                pltpu.sync_copy(rows_v[p], out_h[p].at[pl.ds(base, chunk)])
            return 0

        lax.fori_loop(0, nloc, body, 0)

    return k(*tables, *idxs)


def _sc_gather_scalars(tables, idxs, chunk=640):
    np_ = len(tables)
    etot = idxs[0].shape[0]
    assert etot % chunk == 0 and chunk % _L == 0
    nch = etot // chunk

    scratch = ([pltpu.VMEM(t.shape, t.dtype) for t in tables] +
               [pltpu.VMEM((chunk,), jnp.int32) for _ in range(np_)] +
               [pltpu.VMEM((chunk,), t.dtype) for t in tables])
    out_type = [jax.ShapeDtypeStruct((etot,), t.dtype) for t in tables]

    @functools.partial(pl.kernel, out_type=out_type, mesh=_MESH,
                       scratch_types=scratch,
                       compiler_params=pltpu.CompilerParams(
                           needs_layout_passes=False,
                           use_tc_tiling_on_sc=False))
    def k(*refs):
        tab_h = refs[:np_]
        idx_h = refs[np_:2 * np_]
        out_h = refs[2 * np_:3 * np_]
        tab_v = refs[3 * np_:4 * np_]
        idx_v = refs[4 * np_:5 * np_]
        val_v = refs[5 * np_:6 * np_]
        wid = lax.axis_index("s") * _NC + lax.axis_index("c")
        for p in range(np_):
            pltpu.sync_copy(tab_h[p], tab_v[p])
        nloc = (nch - wid + _NW - 1) // _NW

        def body(j, _):
            base = (wid + j * _NW) * chunk
            for p in range(np_):
                pltpu.sync_copy(idx_h[p].at[pl.ds(base, chunk)], idx_v[p])
            for p in range(np_):
                for g in range(chunk // _L):
                    iv = idx_v[p][pl.ds(g * _L, _L)]
                    val_v[p][pl.ds(g * _L, _L)] = plsc.load_gather(tab_v[p], [iv])
                pltpu.sync_copy(val_v[p], out_h[p].at[pl.ds(base, chunk)])
            return 0

        lax.fori_loop(0, nloc, body, 0)

    return k(*tables, *idxs)


def _sc_scatter_scalars(vals, dst, nseg, chunk=640):
    nv = len(vals)
    etot = dst.shape[0]
    assert etot % chunk == 0 and chunk % _L == 0 and nseg % _L == 0
    nch = etot // chunk

    scratch = ([pltpu.VMEM((nseg,), jnp.float32) for _ in range(nv)] +
               [pltpu.VMEM((chunk,), jnp.int32)] +
               [pltpu.VMEM((chunk,), jnp.float32) for _ in range(nv)])
    out_type = jax.ShapeDtypeStruct((_NW, nv, nseg), jnp.float32)

    @functools.partial(pl.kernel, out_type=out_type, mesh=_MESH,
                       scratch_types=scratch,
                       compiler_params=pltpu.CompilerParams(
                           needs_layout_passes=False,
                           use_tc_tiling_on_sc=False))
    def k(*refs):
        val_h = refs[:nv]
        dst_h = refs[nv]
        out_h = refs[nv + 1]
        acc_v = refs[nv + 2:nv + 2 + nv]
        dst_v = refs[nv + 2 + nv]
        val_v = refs[nv + 3 + nv:nv + 3 + 2 * nv]
        wid = lax.axis_index("s") * _NC + lax.axis_index("c")

        def zero(i, _):
            for a in acc_v:
                a[pl.ds(i * _L, _L)] = jnp.zeros((_L,), jnp.float32)
            return 0

        lax.fori_loop(0, nseg // _L, zero, 0)
        nloc = (nch - wid + _NW - 1) // _NW

        def body(j, _):
            base = (wid + j * _NW) * chunk
            pltpu.sync_copy(dst_h.at[pl.ds(base, chunk)], dst_v)
            for p in range(nv):
                pltpu.sync_copy(val_h[p].at[pl.ds(base, chunk)], val_v[p])
            for g in range(chunk // _L):
                dv = dst_v[pl.ds(g * _L, _L)]
                for p in range(nv):
                    plsc.addupdate_scatter(acc_v[p], [dv],
                                           val_v[p][pl.ds(g * _L, _L)])
            return 0

        lax.fori_loop(0, nloc, body, 0)
        for p in range(nv):
            pltpu.sync_copy(acc_v[p], out_h.at[wid, p])

    return k(*vals, dst)


def _sc_scatter_rows(rows, dst, zeros, nseg, chunk=128):
    _, etot, w = rows.shape
    assert etot % chunk == 0 and chunk <= 128 and nseg % _NS == 0
    assert w == _WC
    nch = etot // chunk
    rows_per_tile = nseg // _NS

    scratch = [pltpu.VMEM_SHARED((nseg, _WC), jnp.float32),
               pltpu.VMEM((chunk,), jnp.int32),
               pltpu.VMEM((chunk, _WC), jnp.float32)]
    out_type = jax.ShapeDtypeStruct((_NC, nseg, _WC), jnp.float32)

    @functools.partial(pl.kernel, out_type=out_type, mesh=_MESH,
                       scratch_types=scratch,
                       compiler_params=pltpu.CompilerParams(
                           needs_layout_passes=False,
                           use_tc_tiling_on_sc=False))
    def k(rows_h, dst_h, zeros_h, out_h, acc_s, dst_v, rows_v):
        cid = lax.axis_index("c")
        sid = lax.axis_index("s")
        row0 = sid * rows_per_tile
        pltpu.sync_copy(zeros_h.at[pl.ds(row0, rows_per_tile)],
                        acc_s.at[pl.ds(row0, rows_per_tile)])
        plsc.subcore_barrier()
        nloc = (nch - sid + _NS - 1) // _NS

        def body(j, _):
            base = (sid + j * _NS) * chunk
            pltpu.sync_copy(dst_h.at[pl.ds(base, chunk)], dst_v)
            pltpu.sync_copy(rows_h.at[cid, pl.ds(base, chunk)], rows_v)
            pltpu.sync_copy(rows_v, acc_s.at[dst_v], add=True)
            return 0

        lax.fori_loop(0, nloc, body, 0)
        plsc.subcore_barrier()
        pltpu.sync_copy(acc_s.at[pl.ds(row0, rows_per_tile)],
                        out_h.at[cid, pl.ds(row0, rows_per_tile)])

    return k(rows, dst, zeros)


def _score_body(qg, krtg, vrtg, krog, vrog, wrows):
    q = qg[...]
    et = jnp.exp(jnp.sum(krtg[...] * q, axis=1, keepdims=True))
    eo = jnp.exp(jnp.sum(krog[...] * q, axis=1, keepdims=True))
    ones = jnp.ones_like(et)
    pad = jnp.zeros((et.shape[0], 5), jnp.float32)
    blk_a = jnp.concatenate([et * vrtg[...], et, eo, ones, pad], axis=1)
    blk_b = jnp.concatenate([eo * vrog[...], et, eo, ones, pad], axis=1)
    wrows[...] = jnp.stack([blk_a, blk_b])


def _score_stage(qg, krtg, vrtg, krog, vrog):
    be = 2000
    grid = (E // be,)
    in_specs = [pl.BlockSpec((be, D), lambda i: (i, 0))] * 5
    return pl.pallas_call(
        _score_body, grid=grid, in_specs=in_specs,
        out_specs=pl.BlockSpec((2, be, _WC), lambda i: (0, i, 0)),
        out_shape=jax.ShapeDtypeStruct((2, E, _WC), jnp.float32),
    )(qg, krtg, vrtg, krog, vrog)


def _prep_body(ha2g, rag, ebg, rbg, eproj, rsum, geacc):
    ra = rag[...]
    rb = rbg[...]
    eproj[...] = ra[:, 30:31] + rb[:, 30:31]
    rsum[...] = ra + rb
    z = ha2g[...] + ebg[...]
    ez = jnp.exp(z)
    nm = jnp.sum(ez * z, axis=0, keepdims=True)
    dn = jnp.sum(ez, axis=0, keepdims=True)
    blk = jnp.concatenate([nm, dn], axis=0).reshape(1, 2, D)

    @pl.when(pl.program_id(0) == 0)
    def _():
        geacc[...] = blk

    @pl.when(pl.program_id(0) != 0)
    def _():
        geacc[...] += blk


def _prep_stage(ha2g, rag, ebg, rbg):
    be = 2000
    grid = (E // be,)
    in_specs = [pl.BlockSpec((be, D), lambda i: (i, 0)),
                pl.BlockSpec((be, 32), lambda i: (i, 0)),
                pl.BlockSpec((be, D), lambda i: (i, 0)),
                pl.BlockSpec((be, 32), lambda i: (i, 0))]
    out_specs = [pl.BlockSpec((be, 1), lambda i: (i, 0)),
                 pl.BlockSpec((be, 32), lambda i: (i, 0)),
                 pl.BlockSpec((1, 2, D), lambda i: (0, 0, 0))]
    out_shape = [jax.ShapeDtypeStruct((E, 1), jnp.float32),
                 jax.ShapeDtypeStruct((E, 32), jnp.float32),
                 jax.ShapeDtypeStruct((1, 2, D), jnp.float32)]
    return pl.pallas_call(
        _prep_body, grid=grid, in_specs=in_specs, out_specs=out_specs,
        out_shape=out_shape)(ha2g, rag, ebg, rbg)


def _node_body(aggp, x_node, awn, abn, glw, grw, scal, xl, xr, cnt_o, gnp):
    a = aggp[...]
    num_t = a[0][:, :D]
    num_o = a[1][:, :D]
    den_t = a[0][:, D:D + 1]
    den_o = a[1][:, D + 1:D + 2]
    cnt_o[...] = a[0][:, D + 2:D + 3]
    agg = (num_t / (den_t + 1e-16) +
           num_o / (den_o + 1e-16))
    a_n = scal[0, 0]
    o = _gelu(agg) @ awn[...] + abn[...]
    hid = a_n * o + (1.0 - a_n) * x_node[...]
    xl[...] = hid @ glw[...] + scal[0, 1]
    xr[...] = hid @ grw[...] + scal[0, 2]
    mb = jnp.max(hid, axis=0, keepdims=True)
    @pl.when(pl.program_id(0) == 0)
    def _():
        ez = jnp.exp(hid - mb)
        gnp[...] = jnp.concatenate(
            [mb, jnp.sum(ez * hid, axis=0, keepdims=True),
             jnp.sum(ez, axis=0, keepdims=True)], axis=0)

    @pl.when(pl.program_id(0) != 0)
    def _():
        g = gnp[...]
        m_old = g[0:1]
        m_new = jnp.maximum(m_old, mb)
        ez = jnp.exp(hid - m_new)
        sc = jnp.exp(m_old - m_new)
        gnp[...] = jnp.concatenate(
            [m_new,
             g[1:2] * sc + jnp.sum(ez * hid, axis=0, keepdims=True),
             g[2:3] * sc + jnp.sum(ez, axis=0, keepdims=True)], axis=0)


def _node_stage(agg_partials, x_node, prm):
    bn = 2000
    scal = jnp.stack([jax.nn.sigmoid(prm['skip_node']),
                      prm['gat_l_b'][0], prm['gat_r_b'][0]]).reshape(1, 3)

    def rep(shape):
        return pl.BlockSpec(shape, lambda i: (0,) * len(shape))

    return pl.pallas_call(
        _node_body, grid=(N // bn,),
        in_specs=[pl.BlockSpec((2, bn, _WC), lambda i: (0, i, 0)),
                  pl.BlockSpec((bn, D), lambda i: (i, 0)),
                  rep((D, D)), rep((1, D)), rep((D, 1)), rep((D, 1)),
                  rep((1, 3))],
        out_specs=[pl.BlockSpec((bn, 1), lambda i: (i, 0)),
                   pl.BlockSpec((bn, 1), lambda i: (i, 0)),
                   pl.BlockSpec((bn, 1), lambda i: (i, 0)),
                   pl.BlockSpec((3, D), lambda i: (0, 0))],
        out_shape=[jax.ShapeDtypeStruct((N, 1), jnp.float32),
                   jax.ShapeDtypeStruct((N, 1), jnp.float32),
                   jax.ShapeDtypeStruct((N, 1), jnp.float32),
                   jax.ShapeDtypeStruct((3, D), jnp.float32)],
    )(agg_partials, x_node, prm['a_w_node'], prm['a_b_node'].reshape(1, D),
      prm['gat_l_w'], prm['gat_r_w'], scal)


def _gat_edge_body(xls, xrd, ep, scal, es, esx):
    z = xls[...] + xrd[...] + ep[...]
    s = jnp.maximum(z, 0.2 * z) * scal[0, 0]
    e = jnp.exp(s)
    es[...] = e
    esx[...] = e * xls[...]


def _gat_edge_stage(xls, xrd, eproj, prm):
    be = 2000
    scal = prm['gat_att'].reshape(1, 1)
    return pl.pallas_call(
        _gat_edge_body, grid=(E // be,),
        in_specs=[pl.BlockSpec((be, 1), lambda i: (i, 0))] * 3 +
                 [pl.BlockSpec((1, 1), lambda i: (0, 0))],
        out_specs=[pl.BlockSpec((be, 1), lambda i: (i, 0))] * 2,
        out_shape=[jax.ShapeDtypeStruct((E, 1), jnp.float32)] * 2,
    )(xls.reshape(E, 1), xrd.reshape(E, 1), eproj, scal)


def _rootframe_body(gatp, cntc, xl, xr, gnp, gep, fw, fb, rwb, rb, scal,
                    root_preds, amax, frame, const32):
    g = jnp.sum(gatp[...], axis=0)
    cnt = cntc[...].reshape(1, N)
    att = scal[0, 0]
    bias = scal[0, 1]
    xlr = xl[...].reshape(1, N)
    xrr = xr[...].reshape(1, N)
    loop_eproj = g[0:1] / jnp.maximum(cnt, 1.0)
    z = xlr + xrr + loop_eproj
    s_self = jnp.maximum(z, 0.2 * z) * att
    es = jnp.exp(s_self)
    den = g[1:2] + es
    num = g[2:3] + es * xlr
    root = num / (den + 1e-16) + bias
    m = jnp.max(root, axis=1, keepdims=True)
    e = jnp.exp(root - m)
    lse = jnp.log(jnp.sum(e, axis=1, keepdims=True))
    root_preds[...] = root - m - lse
    idx = lax.broadcasted_iota(jnp.int32, (1, N), 1)
    amax[...] = jnp.min(jnp.where(root == m, idx, N), axis=1, keepdims=True)
    num2 = jnp.sum(gep[...], axis=0)
    ge = num2[0:1] / num2[1:2]
    gg = gnp[...]
    gn = gg[1:2] / gg[2:3]
    grep = jnp.concatenate([gn, ge], axis=1)
    f = grep @ fw[...] + fb[...]
    mf = jnp.max(f, axis=1, keepdims=True)
    lsef = jnp.log(jnp.sum(jnp.exp(f - mf), axis=1, keepdims=True))
    frame[...] = f - mf - lsef
    const32[...] = gn @ rwb[...] + rb[...]


def _rootframe_stage(gat_partials, cntc, xl, xr, gnp, ge_partials, prm,
                     rwb_pad, rb_pad):
    scal = jnp.stack([prm['gat_att'][0], prm['gat_bias'][0]]).reshape(1, 2)
    return pl.pallas_call(
        _rootframe_body,
        out_shape=[jax.ShapeDtypeStruct((1, N), jnp.float32),
                   jax.ShapeDtypeStruct((1, 1), jnp.int32),
                   jax.ShapeDtypeStruct((1, NF), jnp.float32),
                   jax.ShapeDtypeStruct((1, 32), jnp.float32)],
    )(gat_partials, cntc, xl, xr, gnp, ge_partials,
      prm['frame_w'], prm['frame_b'].reshape(1, NF), rwb_pad, rb_pad, scal)


def _role_body(rsum, src, amax, const32, out):
    r = rsum[...] + const32[...]
    keep = src[...] == amax[0, 0]
    r = jnp.where(keep, r, 0.0)
    lane = lax.broadcasted_iota(jnp.int32, r.shape, 1)
    valid = lane < NR
    rm = jnp.where(valid, r, _NEG)
    m = jnp.max(rm, axis=1, keepdims=True)
    e = jnp.where(valid, jnp.exp(r - m), 0.0)
    lse = jnp.log(jnp.sum(e, axis=1, keepdims=True))
    out[...] = r - m - lse


def _role_stage(rsum, edge_src, amax, const32):
    be = 2000
    grid = (E // be,)
    return pl.pallas_call(
        _role_body, grid=grid,
        in_specs=[pl.BlockSpec((be, 32), lambda i: (i, 0)),
                  pl.BlockSpec((be, 1), lambda i: (i, 0)),
                  pl.BlockSpec((1, 1), lambda i: (0, 0)),
                  pl.BlockSpec((1, 32), lambda i: (0, 0))],
        out_specs=pl.BlockSpec((be, 32), lambda i: (i, 0)),
        out_shape=jax.ShapeDtypeStruct((E, 32), jnp.float32),
    )(rsum, edge_src.reshape(E, 1), amax, const32)


def kernel(node_x, edge_x, edge_src, edge_dst, params):
    prm = params
    emb = prm['pred_emb']
    a_e = jax.nn.sigmoid(prm['skip_edge'])
    rwt2 = jnp.concatenate(
        [prm['role_w'][:D], prm['gat_e_w'],
         jnp.zeros((D, 32 - NR - 1), jnp.float32)], axis=1)
    rwb_pad = jnp.pad(prm['role_w'][D:], ((0, 0), (0, 32 - NR)))
    rb_pad = jnp.pad(prm['role_b'], (0, 32 - NR)).reshape(1, 32)

    q_tab, mcs, mce = _make_tables(emb, prm, a_e, rwt2)

    (x_node,) = _sc_gather_rows([emb], [node_x], chunk=80)

    (krtg, ha2g, vrtg, rag, krog, ebg, vrog, rbg, qg) = _sc_big_gather(
        mcs, mce, q_tab, node_x, edge_src, edge_dst, edge_x)

    wrows = _score_stage(qg, krtg, vrtg, krog, vrog)
    eproj, rsum, ge_partials = _prep_stage(ha2g, rag, ebg, rbg)

    zeros_nw = jnp.zeros((N, _WC), jnp.float32)
    agg_partials = _sc_scatter_rows(wrows, edge_dst, zeros_nw, N)

    xl, xr, cntc, gnp = _node_stage(agg_partials, x_node, prm)

    xls, xrd = _sc_gather_scalars([xl.reshape(N), xr.reshape(N)],
                                  [edge_src, edge_dst])
    es2, esx2 = _gat_edge_stage(xls, xrd, eproj, prm)
    gat_partials = _sc_scatter_scalars(
        [eproj.reshape(E), es2.reshape(E), esx2.reshape(E)], edge_dst, N)

    root_preds2, amax, frame2, const32 = _rootframe_stage(
        gat_partials, cntc, xl, xr, gnp, ge_partials, prm, rwb_pad, rb_pad)

    role32 = _role_stage(rsum, edge_src, amax, const32)

    root_preds = root_preds2.reshape(N)
    frame_preds = frame2.reshape(NF)
    role_preds = role32[:, :NR]
    return ((root_preds, frame_preds), role_preds)


# speedup vs baseline: 1.2875x; 1.0802x over previous
import functools

import jax
import jax.numpy as jnp
from jax import lax
from jax.experimental import pallas as pl
from jax.experimental.pallas import tpu as pltpu
from jax.experimental.pallas import tpu_sc as plsc

_NC = 2
_NS = 16
_NW = _NC * _NS
_L = 16

_MESH = plsc.VectorSubcoreMesh(core_axis_name="c", subcore_axis_name="s",
                               num_cores=_NC, num_subcores=_NS)

N = 10000
E = 160000
D = 128
P = 20000
NF = 1200
NR = 30

_W = 3 * D + 32

_NEG = -1e30


def _erf(x):
    s = jnp.sign(x)
    a = jnp.abs(x)
    t = 1.0 / (1.0 + 0.3275911 * a)
    poly = t * (0.254829592 + t * (-0.284496736 + t * (1.421413741 +
           t * (-1.453152027 + t * 1.061405429))))
    return s * (1.0 - poly * jnp.exp(-a * a))


def _gelu(x):
    return 0.5 * x * (1.0 + _erf(x * 0.7071067811865476))


def _tables_body(emb, kwn, kbn, qwn, qbn, vwn, vbn, art, mrt, mri,
                 kwe, kbe, aro, vwe, vbe, mro, awe, abe, rwt2,
                 scal,
                 q_tab, mcs, mce):
    x = emb[...]
    a_e = scal[0, 0]
    ct = scal[0, 1]
    co = scal[0, 2]
    q_tab[...] = x @ qwn[...] + qbn[...]
    krt = (x @ (kwn[...] @ art[...]) + kbn[...] @ art[...]) * ct
    vrt = x @ (vwn[...] @ mrt[...]) + vbn[...] @ mrt[...]
    kro = (x @ (kwe[...] @ aro[...]) + kbe[...] @ aro[...]) * co
    vro = x @ (vwe[...] @ mro[...]) + vbe[...] @ mro[...]
    vrin = x @ (vwn[...] @ mri[...]) + vbn[...] @ mri[...]
    hidA = _gelu(vrin) @ awe[...] + abe[...]
    hidA2 = a_e * hidA
    embB = (1.0 - a_e) * x
    mcs[...] = jnp.concatenate([krt, hidA2, vrt, hidA2 @ rwt2[...]], axis=1)
    mce[...] = jnp.concatenate([kro, embB, vro, embB @ rwt2[...]], axis=1)


def _make_tables(emb, prm, a_e, rwt2):
    bp = 2000
    grid = (P // bp,)
    scal = jnp.stack([a_e,
                      prm['p_rel_true'] / jnp.sqrt(jnp.float32(D)),
                      prm['p_rel_out'] / jnp.sqrt(jnp.float32(D))]).reshape(1, 3)

    def rep(shape):
        return pl.BlockSpec(shape, lambda i: (0,) * len(shape))

    dd = rep((D, D))
    db = rep((1, D))
    in_specs = [pl.BlockSpec((bp, D), lambda i: (i, 0)),
                dd, db, dd, db, dd, db, dd, dd, dd,
                dd, db, dd, dd, db, dd, dd, db, rep((D, 32)),
                rep((1, 3))]
    out_specs = [pl.BlockSpec((bp, D), lambda i: (i, 0)),
                 pl.BlockSpec((bp, _W), lambda i: (i, 0)),
                 pl.BlockSpec((bp, _W), lambda i: (i, 0))]
    out_shape = [jax.ShapeDtypeStruct((P, D), jnp.float32),
                 jax.ShapeDtypeStruct((P, _W), jnp.float32),
                 jax.ShapeDtypeStruct((P, _W), jnp.float32)]
    args = (emb,
            prm['k_w_node'], prm['k_b_node'].reshape(1, D),
            prm['q_w_node'], prm['q_b_node'].reshape(1, D),
            prm['v_w_node'], prm['v_b_node'].reshape(1, D),
            prm['a_rel_true'], prm['m_rel_true'], prm['m_rel_in'],
            prm['k_w_edge'], prm['k_b_edge'].reshape(1, D),
            prm['a_rel_out'],
            prm['v_w_edge'], prm['v_b_edge'].reshape(1, D),
            prm['m_rel_out'],
            prm['a_w_edge'], prm['a_b_edge'].reshape(1, D),
            rwt2, scal)
    return pl.pallas_call(
        _tables_body, grid=grid, in_specs=in_specs, out_specs=out_specs,
        out_shape=out_shape)(*args)


def _sc_big_gather(mcs, mce, q_tab, node_x, edge_src, edge_dst, edge_x,
                   chunk=64):
    nch = E // chunk
    assert E % chunk == 0 and chunk % _L == 0 and chunk <= 128

    scratch = [pltpu.VMEM((N,), jnp.int32),
               pltpu.VMEM((chunk,), jnp.int32),
               pltpu.VMEM((chunk,), jnp.int32),
               pltpu.VMEM((chunk,), jnp.int32),
               pltpu.VMEM((chunk,), jnp.int32),
               pltpu.VMEM((chunk,), jnp.int32),
               pltpu.VMEM((chunk, _W), jnp.float32),
               pltpu.VMEM((chunk, _W), jnp.float32),
               pltpu.VMEM((chunk, D), jnp.float32),
               pltpu.SemaphoreType.DMA]
    fd = jax.ShapeDtypeStruct((E, D), jnp.float32)
    f32 = jax.ShapeDtypeStruct((E, 32), jnp.float32)
    out_type = [fd, fd, fd, f32, fd, fd, fd, f32, fd]

    @functools.partial(pl.kernel, out_type=out_type, mesh=_MESH,
                       scratch_types=scratch,
                       compiler_params=pltpu.CompilerParams(
                           needs_layout_passes=False,
                           use_tc_tiling_on_sc=False))
    def k(mcs_h, mce_h, q_h, nx_h, es_h, ed_h, ex_h,
          krt_h, ha2_h, vrt_h, ra_h, kro_h, eb_h, vro_h, rb_h, oq_h,
          nx_v, es_v, ed_v, ex_v, cs_v, cd_v, bcs_v, bce_v, bq_v, sem):
        wid = lax.axis_index("s") * _NC + lax.axis_index("c")
        pltpu.sync_copy(nx_h, nx_v)
        nloc = (nch - wid + _NW - 1) // _NW
        rows = pl.ds(0, chunk)

        def body(j, _):
            base = (wid + j * _NW) * chunk
            pltpu.sync_copy(es_h.at[pl.ds(base, chunk)], es_v)
            pltpu.sync_copy(ed_h.at[pl.ds(base, chunk)], ed_v)
            pltpu.sync_copy(ex_h.at[pl.ds(base, chunk)], ex_v)
            for g in range(chunk // _L):
                sl = pl.ds(g * _L, _L)
                cs_v[sl] = plsc.load_gather(nx_v, [es_v[sl]])
                cd_v[sl] = plsc.load_gather(nx_v, [ed_v[sl]])
            h1 = pltpu.async_copy(mcs_h.at[cs_v], bcs_v, sem)
            h2 = pltpu.async_copy(mce_h.at[ex_v], bce_v, sem)
            h3 = pltpu.async_copy(q_h.at[cd_v], bq_v, sem)
            h1.wait()
            h2.wait()
            h3.wait()
            dst = pl.ds(base, chunk)
            pltpu.sync_copy(bcs_v.at[rows, pl.ds(0, D)], krt_h.at[dst])
            pltpu.sync_copy(bcs_v.at[rows, pl.ds(D, D)], ha2_h.at[dst])
            pltpu.sync_copy(bcs_v.at[rows, pl.ds(2 * D, D)], vrt_h.at[dst])
            pltpu.sync_copy(bcs_v.at[rows, pl.ds(3 * D, 32)], ra_h.at[dst])
            pltpu.sync_copy(bce_v.at[rows, pl.ds(0, D)], kro_h.at[dst])
            pltpu.sync_copy(bce_v.at[rows, pl.ds(D, D)], eb_h.at[dst])
            pltpu.sync_copy(bce_v.at[rows, pl.ds(2 * D, D)], vro_h.at[dst])
            pltpu.sync_copy(bce_v.at[rows, pl.ds(3 * D, 32)], rb_h.at[dst])
            pltpu.sync_copy(bq_v, oq_h.at[dst])
            return 0

        lax.fori_loop(0, nloc, body, 0)

    return k(mcs, mce, q_tab, node_x, edge_src, edge_dst, edge_x)


def _sc_gather_rows(tables, idxs, chunk=128):
    np_ = len(tables)
    etot = idxs[0].shape[0]
    assert etot % chunk == 0 and chunk % _L == 0 and chunk <= 128
    nch = etot // chunk

    scratch = ([pltpu.VMEM((chunk,), jnp.int32) for _ in range(np_)] +
               [pltpu.VMEM((chunk, t.shape[1]), t.dtype) for t in tables] +
               [pltpu.SemaphoreType.DMA])
    out_type = [jax.ShapeDtypeStruct((etot, t.shape[1]), t.dtype)
                for t in tables]

    @functools.partial(pl.kernel, out_type=out_type, mesh=_MESH,
                       scratch_types=scratch,
                       compiler_params=pltpu.CompilerParams(
                           needs_layout_passes=False,
                           use_tc_tiling_on_sc=False))
    def k(*refs):
        tab_h = refs[:np_]
        idx_h = refs[np_:2 * np_]
        out_h = refs[2 * np_:3 * np_]
        idx_v = refs[3 * np_:4 * np_]
        rows_v = refs[4 * np_:5 * np_]
        sem = refs[5 * np_]
        wid = lax.axis_index("s") * _NC + lax.axis_index("c")
        nloc = (nch - wid + _NW - 1) // _NW

        def body(j, _):
            base = (wid + j * _NW) * chunk
            for p in range(np_):
                pltpu.sync_copy(idx_h[p].at[pl.ds(base, chunk)], idx_v[p])
            handles = [pltpu.async_copy(tab_h[p].at[idx_v[p]], rows_v[p], sem)
                       for p in range(np_)]
            for h in handles:
                h.wait()
            for p in range(np_):
                pltpu.sync_copy(rows_v[p], out_h[p].at[pl.ds(base, chunk)])
            return 0

        lax.fori_loop(0, nloc, body, 0)

    return k(*tables, *idxs)


def _sc_gather_scalars(tables, idxs, chunk=640):
    np_ = len(tables)
    etot = idxs[0].shape[0]
    assert etot % chunk == 0 and chunk % _L == 0
    nch = etot // chunk

    scratch = ([pltpu.VMEM(t.shape, t.dtype) for t in tables] +
               [pltpu.VMEM((chunk,), jnp.int32) for _ in range(np_)] +
               [pltpu.VMEM((chunk,), t.dtype) for t in tables])
    out_type = [jax.ShapeDtypeStruct((etot,), t.dtype) for t in tables]

    @functools.partial(pl.kernel, out_type=out_type, mesh=_MESH,
                       scratch_types=scratch,
                       compiler_params=pltpu.CompilerParams(
                           needs_layout_passes=False,
                           use_tc_tiling_on_sc=False))
    def k(*refs):
        tab_h = refs[:np_]
        idx_h = refs[np_:2 * np_]
        out_h = refs[2 * np_:3 * np_]
        tab_v = refs[3 * np_:4 * np_]
        idx_v = refs[4 * np_:5 * np_]
        val_v = refs[5 * np_:6 * np_]
        wid = lax.axis_index("s") * _NC + lax.axis_index("c")
        for p in range(np_):
            pltpu.sync_copy(tab_h[p], tab_v[p])
        nloc = (nch - wid + _NW - 1) // _NW

        def body(j, _):
            base = (wid + j * _NW) * chunk
            for p in range(np_):
                pltpu.sync_copy(idx_h[p].at[pl.ds(base, chunk)], idx_v[p])
            for p in range(np_):
                for g in range(chunk // _L):
                    iv = idx_v[p][pl.ds(g * _L, _L)]
                    val_v[p][pl.ds(g * _L, _L)] = plsc.load_gather(tab_v[p], [iv])
                pltpu.sync_copy(val_v[p], out_h[p].at[pl.ds(base, chunk)])
            return 0

        lax.fori_loop(0, nloc, body, 0)

    return k(*tables, *idxs)


def _sc_scatter_scalars(vals, dst, nseg, count=False, chunk=640):
    nv = len(vals)
    nacc = nv + (1 if count else 0)
    etot = dst.shape[0]
    assert etot % chunk == 0 and chunk % _L == 0 and nseg % _L == 0
    nch = etot // chunk

    scratch = ([pltpu.VMEM((nseg,), jnp.float32) for _ in range(nacc)] +
               [pltpu.VMEM((chunk,), jnp.int32)] +
               [pltpu.VMEM((chunk,), jnp.float32) for _ in range(nv)])
    out_type = jax.ShapeDtypeStruct((_NW, nacc, nseg), jnp.float32)

    @functools.partial(pl.kernel, out_type=out_type, mesh=_MESH,
                       scratch_types=scratch,
                       compiler_params=pltpu.CompilerParams(
                           needs_layout_passes=False,
                           use_tc_tiling_on_sc=False))
    def k(*refs):
        val_h = refs[:nv]
        dst_h = refs[nv]
        out_h = refs[nv + 1]
        acc_v = refs[nv + 2:nv + 2 + nacc]
        dst_v = refs[nv + 2 + nacc]
        val_v = refs[nv + 3 + nacc:nv + 3 + nacc + nv]
        wid = lax.axis_index("s") * _NC + lax.axis_index("c")

        def zero(i, _):
            for a in acc_v:
                a[pl.ds(i * _L, _L)] = jnp.zeros((_L,), jnp.float32)
            return 0

        lax.fori_loop(0, nseg // _L, zero, 0)
        nloc = (nch - wid + _NW - 1) // _NW

        def body(j, _):
            base = (wid + j * _NW) * chunk
            pltpu.sync_copy(dst_h.at[pl.ds(base, chunk)], dst_v)
            for p in range(nv):
                pltpu.sync_copy(val_h[p].at[pl.ds(base, chunk)], val_v[p])
            for g in range(chunk // _L):
                dv = dst_v[pl.ds(g * _L, _L)]
                for p in range(nv):
                    plsc.addupdate_scatter(acc_v[p], [dv],
                                           val_v[p][pl.ds(g * _L, _L)])
                if count:
                    plsc.addupdate_scatter(acc_v[nv], [dv],
                                           jnp.ones((_L,), jnp.float32))
            return 0

        lax.fori_loop(0, nloc, body, 0)
        for p in range(nacc):
            pltpu.sync_copy(acc_v[p], out_h.at[wid, p])

    return k(*vals, dst)


def _sc_scatter_rows(rows, dst, zeros, nseg, chunk=128):
    etot, w = rows.shape
    assert etot % chunk == 0 and chunk <= 128 and nseg % _NS == 0
    nch = etot // chunk
    rows_per_tile = nseg // _NS

    scratch = [pltpu.VMEM_SHARED((nseg, w), jnp.float32),
               pltpu.VMEM((chunk,), jnp.int32),
               pltpu.VMEM((chunk, w), jnp.float32)]
    out_type = jax.ShapeDtypeStruct((_NC, nseg, w), jnp.float32)

    @functools.partial(pl.kernel, out_type=out_type, mesh=_MESH,
                       scratch_types=scratch,
                       compiler_params=pltpu.CompilerParams(
                           needs_layout_passes=False,
                           use_tc_tiling_on_sc=False))
    def k(rows_h, dst_h, zeros_h, out_h, acc_s, dst_v, rows_v):
        cid = lax.axis_index("c")
        sid = lax.axis_index("s")
        wid = sid * _NC + cid
        row0 = sid * rows_per_tile
        pltpu.sync_copy(zeros_h.at[pl.ds(row0, rows_per_tile)],
                        acc_s.at[pl.ds(row0, rows_per_tile)])
        plsc.subcore_barrier()
        nloc = (nch - wid + _NW - 1) // _NW

        def body(j, _):
            base = (wid + j * _NW) * chunk
            pltpu.sync_copy(dst_h.at[pl.ds(base, chunk)], dst_v)
            pltpu.sync_copy(rows_h.at[pl.ds(base, chunk)], rows_v)
            pltpu.sync_copy(rows_v, acc_s.at[dst_v], add=True)
            return 0

        lax.fori_loop(0, nloc, body, 0)
        plsc.subcore_barrier()
        pltpu.sync_copy(acc_s.at[pl.ds(row0, rows_per_tile)],
                        out_h.at[cid, pl.ds(row0, rows_per_tile)])

    return k(rows, dst, zeros)


def _score_body(qg, krtg, krog, et, eo):
    q = qg[...]
    et[...] = jnp.exp(jnp.sum(krtg[...] * q, axis=1, keepdims=True))
    eo[...] = jnp.exp(jnp.sum(krog[...] * q, axis=1, keepdims=True))


def _score_stage(qg, krtg, krog):
    be = 2000
    grid = (E // be,)
    return pl.pallas_call(
        _score_body, grid=grid,
        in_specs=[pl.BlockSpec((be, D), lambda i: (i, 0))] * 3,
        out_specs=[pl.BlockSpec((be, 1), lambda i: (i, 0))] * 2,
        out_shape=[jax.ShapeDtypeStruct((E, 1), jnp.float32)] * 2,
    )(qg, krtg, krog)


def _combine_body(pp, out):
    out[...] = jnp.sum(pp[...], axis=0)


def _combine_stage(partials):
    k, na, n = partials.shape
    return pl.pallas_call(
        _combine_body,
        out_shape=jax.ShapeDtypeStruct((na, n), jnp.float32),
    )(partials)


def _comb_rows_body(vrtg, vrog, et, eo, dtg, dog, comb):
    at = et[...] / (dtg[...] + 1e-16)
    ao = eo[...] / (dog[...] + 1e-16)
    comb[...] = at * vrtg[...] + ao * vrog[...]


def _comb_rows_stage(vrtg, vrog, et, eo, dtg, dog):
    be = 2000
    return pl.pallas_call(
        _comb_rows_body, grid=(E // be,),
        in_specs=[pl.BlockSpec((be, D), lambda i: (i, 0))] * 2 +
                 [pl.BlockSpec((be, 1), lambda i: (i, 0))] * 4,
        out_specs=pl.BlockSpec((be, D), lambda i: (i, 0)),
        out_shape=jax.ShapeDtypeStruct((E, D), jnp.float32),
    )(vrtg, vrog, et, eo, dtg.reshape(E, 1), dog.reshape(E, 1))


def _prep_body(ha2g, rag, ebg, rbg, eproj, rsum, geacc):
    ra = rag[...]
    rb = rbg[...]
    eproj[...] = ra[:, 30:31] + rb[:, 30:31]
    rsum[...] = ra + rb
    z = ha2g[...] + ebg[...]
    ez = jnp.exp(z)
    nm = jnp.sum(ez * z, axis=0, keepdims=True)
    dn = jnp.sum(ez, axis=0, keepdims=True)
    blk = jnp.concatenate([nm, dn], axis=0).reshape(1, 2, D)

    @pl.when(pl.program_id(0) == 0)
    def _():
        geacc[...] = blk

    @pl.when(pl.program_id(0) != 0)
    def _():
        geacc[...] += blk


def _prep_stage(ha2g, rag, ebg, rbg):
    be = 2000
    grid = (E // be,)
    in_specs = [pl.BlockSpec((be, D), lambda i: (i, 0)),
                pl.BlockSpec((be, 32), lambda i: (i, 0)),
                pl.BlockSpec((be, D), lambda i: (i, 0)),
                pl.BlockSpec((be, 32), lambda i: (i, 0))]
    out_specs = [pl.BlockSpec((be, 1), lambda i: (i, 0)),
                 pl.BlockSpec((be, 32), lambda i: (i, 0)),
                 pl.BlockSpec((1, 2, D), lambda i: (0, 0, 0))]
    out_shape = [jax.ShapeDtypeStruct((E, 1), jnp.float32),
                 jax.ShapeDtypeStruct((E, 32), jnp.float32),
                 jax.ShapeDtypeStruct((1, 2, D), jnp.float32)]
    return pl.pallas_call(
        _prep_body, grid=grid, in_specs=in_specs, out_specs=out_specs,
        out_shape=out_shape)(ha2g, rag, ebg, rbg)


def _node_body(aggp, x_node, awn, abn, glw, grw, scal, xl, xr, gnp):
    a = aggp[...]
    agg = a[0] + a[1]
    a_n = scal[0, 0]
    o = _gelu(agg) @ awn[...] + abn[...]
    hid = a_n * o + (1.0 - a_n) * x_node[...]
    xl[...] = hid @ glw[...] + scal[0, 1]
    xr[...] = hid @ grw[...] + scal[0, 2]
    mb = jnp.max(hid, axis=0, keepdims=True)
    @pl.when(pl.program_id(0) == 0)
    def _():
        ez = jnp.exp(hid - mb)
        gnp[...] = jnp.concatenate(
            [mb, jnp.sum(ez * hid, axis=0, keepdims=True),
             jnp.sum(ez, axis=0, keepdims=True)], axis=0)

    @pl.when(pl.program_id(0) != 0)
    def _():
        g = gnp[...]
        m_old = g[0:1]
        m_new = jnp.maximum(m_old, mb)
        ez = jnp.exp(hid - m_new)
        sc = jnp.exp(m_old - m_new)
        gnp[...] = jnp.concatenate(
            [m_new,
             g[1:2] * sc + jnp.sum(ez * hid, axis=0, keepdims=True),
             g[2:3] * sc + jnp.sum(ez, axis=0, keepdims=True)], axis=0)


def _node_stage(agg_partials, x_node, prm):
    bn = 2000
    scal = jnp.stack([jax.nn.sigmoid(prm['skip_node']),
                      prm['gat_l_b'][0], prm['gat_r_b'][0]]).reshape(1, 3)

    def rep(shape):
        return pl.BlockSpec(shape, lambda i: (0,) * len(shape))

    return pl.pallas_call(
        _node_body, grid=(N // bn,),
        in_specs=[pl.BlockSpec((2, bn, D), lambda i: (0, i, 0)),
                  pl.BlockSpec((bn, D), lambda i: (i, 0)),
                  rep((D, D)), rep((1, D)), rep((D, 1)), rep((D, 1)),
                  rep((1, 3))],
        out_specs=[pl.BlockSpec((bn, 1), lambda i: (i, 0)),
                   pl.BlockSpec((bn, 1), lambda i: (i, 0)),
                   pl.BlockSpec((3, D), lambda i: (0, 0))],
        out_shape=[jax.ShapeDtypeStruct((N, 1), jnp.float32),
                   jax.ShapeDtypeStruct((N, 1), jnp.float32),
                   jax.ShapeDtypeStruct((3, D), jnp.float32)],
    )(agg_partials, x_node, prm['a_w_node'], prm['a_b_node'].reshape(1, D),
      prm['gat_l_w'], prm['gat_r_w'], scal)


def _gat_edge_body(xls, xrd, ep, scal, es, esx):
    z = xls[...] + xrd[...] + ep[...]
    s = jnp.maximum(z, 0.2 * z) * scal[0, 0]
    e = jnp.exp(s)
    es[...] = e
    esx[...] = e * xls[...]


def _gat_edge_stage(xls, xrd, eproj, prm):
    be = 2000
    scal = prm['gat_att'].reshape(1, 1)
    return pl.pallas_call(
        _gat_edge_body, grid=(E // be,),
        in_specs=[pl.BlockSpec((be, 1), lambda i: (i, 0))] * 3 +
                 [pl.BlockSpec((1, 1), lambda i: (0, 0))],
        out_specs=[pl.BlockSpec((be, 1), lambda i: (i, 0))] * 2,
        out_shape=[jax.ShapeDtypeStruct((E, 1), jnp.float32)] * 2,
    )(xls.reshape(E, 1), xrd.reshape(E, 1), eproj, scal)


def _rootframe_body(gatp, cntr, xl, xr, gnp, gep, fw, fb, rwb, rb, scal,
                    root_preds, amax, frame, const32):
    g = jnp.sum(gatp[...], axis=0)
    cnt = cntr[...]
    att = scal[0, 0]
    bias = scal[0, 1]
    xlr = xl[...].reshape(1, N)
    xrr = xr[...].reshape(1, N)
    loop_eproj = g[0:1] / jnp.maximum(cnt, 1.0)
    z = xlr + xrr + loop_eproj
    s_self = jnp.maximum(z, 0.2 * z) * att
    es = jnp.exp(s_self)
    den = g[1:2] + es
    num = g[2:3] + es * xlr
    root = num / (den + 1e-16) + bias
    m = jnp.max(root, axis=1, keepdims=True)
    e = jnp.exp(root - m)
    lse = jnp.log(jnp.sum(e, axis=1, keepdims=True))
    root_preds[...] = root - m - lse
    idx = lax.broadcasted_iota(jnp.int32, (1, N), 1)
    amax[...] = jnp.min(jnp.where(root == m, idx, N), axis=1, keepdims=True)
    num2 = jnp.sum(gep[...], axis=0)
    ge = num2[0:1] / num2[1:2]
    gg = gnp[...]
    gn = gg[1:2] / gg[2:3]
    grep = jnp.concatenate([gn, ge], axis=1)
    f = grep @ fw[...] + fb[...]
    mf = jnp.max(f, axis=1, keepdims=True)
    lsef = jnp.log(jnp.sum(jnp.exp(f - mf), axis=1, keepdims=True))
    frame[...] = f - mf - lsef
    const32[...] = gn @ rwb[...] + rb[...]


def _rootframe_stage(gat_partials, cntr, xl, xr, gnp, ge_partials, prm,
                     rwb_pad, rb_pad):
    scal = jnp.stack([prm['gat_att'][0], prm['gat_bias'][0]]).reshape(1, 2)
    return pl.pallas_call(
        _rootframe_body,
        out_shape=[jax.ShapeDtypeStruct((1, N), jnp.float32),
                   jax.ShapeDtypeStruct((1, 1), jnp.int32),
                   jax.ShapeDtypeStruct((1, NF), jnp.float32),
                   jax.ShapeDtypeStruct((1, 32), jnp.float32)],
    )(gat_partials, cntr, xl, xr, gnp, ge_partials,
      prm['frame_w'], prm['frame_b'].reshape(1, NF), rwb_pad, rb_pad, scal)


def _role_body(rsum, src, amax, const32, out):
    r = rsum[...] + const32[...]
    keep = src[...] == amax[0, 0]
    r = jnp.where(keep, r, 0.0)
    lane = lax.broadcasted_iota(jnp.int32, r.shape, 1)
    valid = lane < NR
    rm = jnp.where(valid, r, _NEG)
    m = jnp.max(rm, axis=1, keepdims=True)
    e = jnp.where(valid, jnp.exp(r - m), 0.0)
    lse = jnp.log(jnp.sum(e, axis=1, keepdims=True))
    out[...] = r - m - lse


def _role_stage(rsum, edge_src, amax, const32):
    be = 2000
    grid = (E // be,)
    return pl.pallas_call(
        _role_body, grid=grid,
        in_specs=[pl.BlockSpec((be, 32), lambda i: (i, 0)),
                  pl.BlockSpec((be, 1), lambda i: (i, 0)),
                  pl.BlockSpec((1, 1), lambda i: (0, 0)),
                  pl.BlockSpec((1, 32), lambda i: (0, 0))],
        out_specs=pl.BlockSpec((be, 32), lambda i: (i, 0)),
        out_shape=jax.ShapeDtypeStruct((E, 32), jnp.float32),
    )(rsum, edge_src.reshape(E, 1), amax, const32)


def kernel(node_x, edge_x, edge_src, edge_dst, params):
    prm = params
    emb = prm['pred_emb']
    a_e = jax.nn.sigmoid(prm['skip_edge'])
    rwt2 = jnp.concatenate(
        [prm['role_w'][:D], prm['gat_e_w'],
         jnp.zeros((D, 32 - NR - 1), jnp.float32)], axis=1)
    rwb_pad = jnp.pad(prm['role_w'][D:], ((0, 0), (0, 32 - NR)))
    rb_pad = jnp.pad(prm['role_b'], (0, 32 - NR)).reshape(1, 32)

    q_tab, mcs, mce = _make_tables(emb, prm, a_e, rwt2)

    (x_node,) = _sc_gather_rows([emb], [node_x], chunk=80)

    (krtg, ha2g, vrtg, rag, krog, ebg, vrog, rbg, qg) = _sc_big_gather(
        mcs, mce, q_tab, node_x, edge_src, edge_dst, edge_x)

    et2, eo2 = _score_stage(qg, krtg, krog)
    den_partials = _sc_scatter_scalars(
        [et2.reshape(E), eo2.reshape(E)], edge_dst, N, count=True)
    dens = _combine_stage(den_partials)
    cntr = dens[2:3]

    eproj, rsum, ge_partials = _prep_stage(ha2g, rag, ebg, rbg)

    dtg, dog = _sc_gather_scalars([dens[0], dens[1]], [edge_dst, edge_dst])
    comb = _comb_rows_stage(vrtg, vrog, et2, eo2, dtg, dog)
    zeros_nd = jnp.zeros((N, D), jnp.float32)
    agg_partials = _sc_scatter_rows(comb, edge_dst, zeros_nd, N)

    xl, xr, gnp = _node_stage(agg_partials, x_node, prm)

    xls, xrd = _sc_gather_scalars([xl.reshape(N), xr.reshape(N)],
                                  [edge_src, edge_dst])
    es2, esx2 = _gat_edge_stage(xls, xrd, eproj, prm)
    gat_partials = _sc_scatter_scalars(
        [eproj.reshape(E), es2.reshape(E), esx2.reshape(E)], edge_dst, N,
        count=False)

    root_preds2, amax, frame2, const32 = _rootframe_stage(
        gat_partials, cntr, xl, xr, gnp, ge_partials, prm, rwb_pad, rb_pad)

    role32 = _role_stage(rsum, edge_src, amax, const32)

    root_preds = root_preds2.reshape(N)
    frame_preds = frame2.reshape(NF)
    role_preds = role32[:, :NR]
    return ((root_preds, frame_preds), role_preds)


# speedup vs baseline: 1.5640x; 1.2148x over previous
import functools

import jax
import jax.numpy as jnp
from jax import lax
from jax.experimental import pallas as pl
from jax.experimental.pallas import tpu as pltpu
from jax.experimental.pallas import tpu_sc as plsc

_NC = 2
_NS = 16
_NW = _NC * _NS
_L = 16

_MESH = plsc.VectorSubcoreMesh(core_axis_name="c", subcore_axis_name="s",
                               num_cores=_NC, num_subcores=_NS)

N = 10000
E = 160000
D = 128
P = 20000
NF = 1200
NR = 30

_W = 3 * D + 32

_NEG = -1e30


def _erf(x):
    s = jnp.sign(x)
    a = jnp.abs(x)
    t = 1.0 / (1.0 + 0.3275911 * a)
    poly = t * (0.254829592 + t * (-0.284496736 + t * (1.421413741 +
           t * (-1.453152027 + t * 1.061405429))))
    return s * (1.0 - poly * jnp.exp(-a * a))


def _gelu(x):
    return 0.5 * x * (1.0 + _erf(x * 0.7071067811865476))


def _tables_body(emb, kwn, kbn, qwn, qbn, vwn, vbn, art, mrt, mri,
                 kwe, kbe, aro, vwe, vbe, mro, awe, abe, rwt2,
                 scal,
                 q_tab, mcs, mce):
    x = emb[...]
    a_e = scal[0, 0]
    ct = scal[0, 1]
    co = scal[0, 2]
    q_tab[...] = x @ qwn[...] + qbn[...]
    krt = (x @ (kwn[...] @ art[...]) + kbn[...] @ art[...]) * ct
    vrt = x @ (vwn[...] @ mrt[...]) + vbn[...] @ mrt[...]
    kro = (x @ (kwe[...] @ aro[...]) + kbe[...] @ aro[...]) * co
    vro = x @ (vwe[...] @ mro[...]) + vbe[...] @ mro[...]
    vrin = x @ (vwn[...] @ mri[...]) + vbn[...] @ mri[...]
    hidA = _gelu(vrin) @ awe[...] + abe[...]
    hidA2 = a_e * hidA
    embB = (1.0 - a_e) * x
    mcs[...] = jnp.concatenate([krt, hidA2, vrt, hidA2 @ rwt2[...]], axis=1)
    mce[...] = jnp.concatenate([kro, embB, vro, embB @ rwt2[...]], axis=1)


def _make_tables(emb, prm, a_e, rwt2):
    bp = 2000
    grid = (P // bp,)
    scal = jnp.stack([a_e,
                      prm['p_rel_true'] / jnp.sqrt(jnp.float32(D)),
                      prm['p_rel_out'] / jnp.sqrt(jnp.float32(D))]).reshape(1, 3)

    def rep(shape):
        return pl.BlockSpec(shape, lambda i: (0,) * len(shape))

    dd = rep((D, D))
    db = rep((1, D))
    in_specs = [pl.BlockSpec((bp, D), lambda i: (i, 0)),
                dd, db, dd, db, dd, db, dd, dd, dd,
                dd, db, dd, dd, db, dd, dd, db, rep((D, 32)),
                rep((1, 3))]
    out_specs = [pl.BlockSpec((bp, D), lambda i: (i, 0)),
                 pl.BlockSpec((bp, _W), lambda i: (i, 0)),
                 pl.BlockSpec((bp, _W), lambda i: (i, 0))]
    out_shape = [jax.ShapeDtypeStruct((P, D), jnp.float32),
                 jax.ShapeDtypeStruct((P, _W), jnp.float32),
                 jax.ShapeDtypeStruct((P, _W), jnp.float32)]
    args = (emb,
            prm['k_w_node'], prm['k_b_node'].reshape(1, D),
            prm['q_w_node'], prm['q_b_node'].reshape(1, D),
            prm['v_w_node'], prm['v_b_node'].reshape(1, D),
            prm['a_rel_true'], prm['m_rel_true'], prm['m_rel_in'],
            prm['k_w_edge'], prm['k_b_edge'].reshape(1, D),
            prm['a_rel_out'],
            prm['v_w_edge'], prm['v_b_edge'].reshape(1, D),
            prm['m_rel_out'],
            prm['a_w_edge'], prm['a_b_edge'].reshape(1, D),
            rwt2, scal)
    return pl.pallas_call(
        _tables_body, grid=grid, in_specs=in_specs, out_specs=out_specs,
        out_shape=out_shape)(*args)


def _sc_big_gather(mcs, mce, q_tab, node_x, edge_src, edge_dst, edge_x,
                   chunk=64):
    nch = E // chunk
    assert E % chunk == 0 and chunk % _L == 0 and chunk <= 128

    scratch = [pltpu.VMEM((N,), jnp.int32),
               pltpu.VMEM((chunk,), jnp.int32),
               pltpu.VMEM((chunk,), jnp.int32),
               pltpu.VMEM((chunk,), jnp.int32),
               pltpu.VMEM((chunk,), jnp.int32),
               pltpu.VMEM((chunk,), jnp.int32),
               pltpu.VMEM((chunk, _W), jnp.float32),
               pltpu.VMEM((chunk, _W), jnp.float32),
               pltpu.VMEM((chunk, D), jnp.float32),
               pltpu.SemaphoreType.DMA]
    fd = jax.ShapeDtypeStruct((E, D), jnp.float32)
    f32 = jax.ShapeDtypeStruct((E, 32), jnp.float32)
    out_type = [fd, fd, fd, f32, fd, fd, fd, f32, fd]

    @functools.partial(pl.kernel, out_type=out_type, mesh=_MESH,
                       scratch_types=scratch,
                       compiler_params=pltpu.CompilerParams(
                           needs_layout_passes=False,
                           use_tc_tiling_on_sc=False))
    def k(mcs_h, mce_h, q_h, nx_h, es_h, ed_h, ex_h,
          krt_h, ha2_h, vrt_h, ra_h, kro_h, eb_h, vro_h, rb_h, oq_h,
          nx_v, es_v, ed_v, ex_v, cs_v, cd_v, bcs_v, bce_v, bq_v, sem):
        wid = lax.axis_index("s") * _NC + lax.axis_index("c")
        pltpu.sync_copy(nx_h, nx_v)
        nloc = (nch - wid + _NW - 1) // _NW
        rows = pl.ds(0, chunk)

        def body(j, _):
            base = (wid + j * _NW) * chunk
            pltpu.sync_copy(es_h.at[pl.ds(base, chunk)], es_v)
            pltpu.sync_copy(ed_h.at[pl.ds(base, chunk)], ed_v)
            pltpu.sync_copy(ex_h.at[pl.ds(base, chunk)], ex_v)
            for g in range(chunk // _L):
                sl = pl.ds(g * _L, _L)
                cs_v[sl] = plsc.load_gather(nx_v, [es_v[sl]])
                cd_v[sl] = plsc.load_gather(nx_v, [ed_v[sl]])
            h1 = pltpu.async_copy(mcs_h.at[cs_v], bcs_v, sem)
            h2 = pltpu.async_copy(mce_h.at[ex_v], bce_v, sem)
            h3 = pltpu.async_copy(q_h.at[cd_v], bq_v, sem)
            h1.wait()
            h2.wait()
            h3.wait()
            dst = pl.ds(base, chunk)
            pltpu.sync_copy(bcs_v.at[rows, pl.ds(0, D)], krt_h.at[dst])
            pltpu.sync_copy(bcs_v.at[rows, pl.ds(D, D)], ha2_h.at[dst])
            pltpu.sync_copy(bcs_v.at[rows, pl.ds(2 * D, D)], vrt_h.at[dst])
            pltpu.sync_copy(bcs_v.at[rows, pl.ds(3 * D, 32)], ra_h.at[dst])
            pltpu.sync_copy(bce_v.at[rows, pl.ds(0, D)], kro_h.at[dst])
            pltpu.sync_copy(bce_v.at[rows, pl.ds(D, D)], eb_h.at[dst])
            pltpu.sync_copy(bce_v.at[rows, pl.ds(2 * D, D)], vro_h.at[dst])
            pltpu.sync_copy(bce_v.at[rows, pl.ds(3 * D, 32)], rb_h.at[dst])
            pltpu.sync_copy(bq_v, oq_h.at[dst])
            return 0

        lax.fori_loop(0, nloc, body, 0)

    return k(mcs, mce, q_tab, node_x, edge_src, edge_dst, edge_x)


def _sc_gather_rows(tables, idxs, chunk=128):
    np_ = len(tables)
    etot = idxs[0].shape[0]
    assert etot % chunk == 0 and chunk % _L == 0 and chunk <= 128
    nch = etot // chunk

    scratch = ([pltpu.VMEM((chunk,), jnp.int32) for _ in range(np_)] +
               [pltpu.VMEM((chunk, t.shape[1]), t.dtype) for t in tables] +
               [pltpu.SemaphoreType.DMA])
    out_type = [jax.ShapeDtypeStruct((etot, t.shape[1]), t.dtype)
                for t in tables]

    @functools.partial(pl.kernel, out_type=out_type, mesh=_MESH,
                       scratch_types=scratch,
                       compiler_params=pltpu.CompilerParams(
                           needs_layout_passes=False,
                           use_tc_tiling_on_sc=False))
    def k(*refs):
        tab_h = refs[:np_]
        idx_h = refs[np_:2 * np_]
        out_h = refs[2 * np_:3 * np_]
        idx_v = refs[3 * np_:4 * np_]
        rows_v = refs[4 * np_:5 * np_]
        sem = refs[5 * np_]
        wid = lax.axis_index("s") * _NC + lax.axis_index("c")
        nloc = (nch - wid + _NW - 1) // _NW

        def body(j, _):
            base = (wid + j * _NW) * chunk
            for p in range(np_):
                pltpu.sync_copy(idx_h[p].at[pl.ds(base, chunk)], idx_v[p])
            handles = [pltpu.async_copy(tab_h[p].at[idx_v[p]], rows_v[p], sem)
                       for p in range(np_)]
            for h in handles:
                h.wait()
            for p in range(np_):
                pltpu.sync_copy(rows_v[p], out_h[p].at[pl.ds(base, chunk)])
            return 0

        lax.fori_loop(0, nloc, body, 0)

    return k(*tables, *idxs)


def _sc_gather_scalars(tables, idxs, chunk=640):
    np_ = len(tables)
    etot = idxs[0].shape[0]
    assert etot % chunk == 0 and chunk % _L == 0
    nch = etot // chunk

    scratch = ([pltpu.VMEM(t.shape, t.dtype) for t in tables] +
               [pltpu.VMEM((chunk,), jnp.int32) for _ in range(np_)] +
               [pltpu.VMEM((chunk,), t.dtype) for t in tables])
    out_type = [jax.ShapeDtypeStruct((etot,), t.dtype) for t in tables]

    @functools.partial(pl.kernel, out_type=out_type, mesh=_MESH,
                       scratch_types=scratch,
                       compiler_params=pltpu.CompilerParams(
                           needs_layout_passes=False,
                           use_tc_tiling_on_sc=False))
    def k(*refs):
        tab_h = refs[:np_]
        idx_h = refs[np_:2 * np_]
        out_h = refs[2 * np_:3 * np_]
        tab_v = refs[3 * np_:4 * np_]
        idx_v = refs[4 * np_:5 * np_]
        val_v = refs[5 * np_:6 * np_]
        wid = lax.axis_index("s") * _NC + lax.axis_index("c")
        for p in range(np_):
            pltpu.sync_copy(tab_h[p], tab_v[p])
        nloc = (nch - wid + _NW - 1) // _NW

        def body(j, _):
            base = (wid + j * _NW) * chunk
            for p in range(np_):
                pltpu.sync_copy(idx_h[p].at[pl.ds(base, chunk)], idx_v[p])
            for p in range(np_):
                for g in range(chunk // _L):
                    iv = idx_v[p][pl.ds(g * _L, _L)]
                    val_v[p][pl.ds(g * _L, _L)] = plsc.load_gather(tab_v[p], [iv])
                pltpu.sync_copy(val_v[p], out_h[p].at[pl.ds(base, chunk)])
            return 0

        lax.fori_loop(0, nloc, body, 0)

    return k(*tables, *idxs)


def _sc_scatter_scalars(vals, dst, nseg, count=False, chunk=640):
    nv = len(vals)
    nacc = nv + (1 if count else 0)
    etot = dst.shape[0]
    assert etot % chunk == 0 and chunk % _L == 0 and nseg % _L == 0
    nch = etot // chunk

    scratch = ([pltpu.VMEM((nseg,), jnp.float32) for _ in range(nacc)] +
               [pltpu.VMEM((chunk,), jnp.int32)] +
               [pltpu.VMEM((chunk,), jnp.float32) for _ in range(nv)])
    out_type = jax.ShapeDtypeStruct((_NW, nacc, nseg), jnp.float32)

    @functools.partial(pl.kernel, out_type=out_type, mesh=_MESH,
                       scratch_types=scratch,
                       compiler_params=pltpu.CompilerParams(
                           needs_layout_passes=False,
                           use_tc_tiling_on_sc=False))
    def k(*refs):
        val_h = refs[:nv]
        dst_h = refs[nv]
        out_h = refs[nv + 1]
        acc_v = refs[nv + 2:nv + 2 + nacc]
        dst_v = refs[nv + 2 + nacc]
        val_v = refs[nv + 3 + nacc:nv + 3 + nacc + nv]
        wid = lax.axis_index("s") * _NC + lax.axis_index("c")

        def zero(i, _):
            for a in acc_v:
                a[pl.ds(i * _L, _L)] = jnp.zeros((_L,), jnp.float32)
            return 0

        lax.fori_loop(0, nseg // _L, zero, 0)
        nloc = (nch - wid + _NW - 1) // _NW

        def body(j, _):
            base = (wid + j * _NW) * chunk
            pltpu.sync_copy(dst_h.at[pl.ds(base, chunk)], dst_v)
            for p in range(nv):
                pltpu.sync_copy(val_h[p].at[pl.ds(base, chunk)], val_v[p])
            for g in range(chunk // _L):
                dv = dst_v[pl.ds(g * _L, _L)]
                for p in range(nv):
                    plsc.addupdate_scatter(acc_v[p], [dv],
                                           val_v[p][pl.ds(g * _L, _L)])
                if count:
                    plsc.addupdate_scatter(acc_v[nv], [dv],
                                           jnp.ones((_L,), jnp.float32))
            return 0

        lax.fori_loop(0, nloc, body, 0)
        for p in range(nacc):
            pltpu.sync_copy(acc_v[p], out_h.at[wid, p])

    return k(*vals, dst)


def _sc_gat_fused(xl, xr, eproj, edge_src, edge_dst, att16, chunk=640):
    nch = E // chunk
    assert E % chunk == 0 and chunk % _L == 0

    scratch = ([pltpu.VMEM((N,), jnp.float32)] * 2 +
               [pltpu.VMEM((_L,), jnp.float32)] +
               [pltpu.VMEM((N,), jnp.float32)] * 3 +
               [pltpu.VMEM((chunk,), jnp.int32)] * 2 +
               [pltpu.VMEM((chunk,), jnp.float32)])
    out_type = jax.ShapeDtypeStruct((_NW, 3, N), jnp.float32)

    @functools.partial(pl.kernel, out_type=out_type, mesh=_MESH,
                       scratch_types=scratch,
                       compiler_params=pltpu.CompilerParams(
                           needs_layout_passes=False,
                           use_tc_tiling_on_sc=False))
    def k(xl_h, xr_h, ep_h, es_h, ed_h, att_h, out_h,
          xl_v, xr_v, att_v, acc0, acc1, acc2, src_v, dst_v, ep_v):
        wid = lax.axis_index("s") * _NC + lax.axis_index("c")
        pltpu.sync_copy(xl_h, xl_v)
        pltpu.sync_copy(xr_h, xr_v)
        pltpu.sync_copy(att_h, att_v)

        def zero(i, _):
            z16 = jnp.zeros((_L,), jnp.float32)
            acc0[pl.ds(i * _L, _L)] = z16
            acc1[pl.ds(i * _L, _L)] = z16
            acc2[pl.ds(i * _L, _L)] = z16
            return 0

        lax.fori_loop(0, N // _L, zero, 0)
        nloc = (nch - wid + _NW - 1) // _NW

        def body(j, _):
            base = (wid + j * _NW) * chunk
            pltpu.sync_copy(es_h.at[pl.ds(base, chunk)], src_v)
            pltpu.sync_copy(ed_h.at[pl.ds(base, chunk)], dst_v)
            pltpu.sync_copy(ep_h.at[pl.ds(base, chunk)], ep_v)
            att = att_v[pl.ds(0, _L)]
            for g in range(chunk // _L):
                sl = pl.ds(g * _L, _L)
                sv = src_v[sl]
                dv = dst_v[sl]
                ep = ep_v[sl]
                xls = plsc.load_gather(xl_v, [sv])
                xrd = plsc.load_gather(xr_v, [dv])
                z = xls + xrd + ep
                e = jnp.exp(jnp.maximum(z, 0.2 * z) * att)
                plsc.addupdate_scatter(acc0, [dv], ep)
                plsc.addupdate_scatter(acc1, [dv], e)
                plsc.addupdate_scatter(acc2, [dv], e * xls)
            return 0

        lax.fori_loop(0, nloc, body, 0)
        pltpu.sync_copy(acc0, out_h.at[wid, 0])
        pltpu.sync_copy(acc1, out_h.at[wid, 1])
        pltpu.sync_copy(acc2, out_h.at[wid, 2])

    return k(xl, xr, eproj, edge_src, edge_dst, att16)


def _sc_scatter_rows(rows, dst, zeros, nseg, chunk=128):
    etot, w = rows.shape
    assert etot % chunk == 0 and chunk <= 128 and nseg % _NS == 0
    nch = etot // chunk
    rows_per_tile = nseg // _NS

    scratch = [pltpu.VMEM_SHARED((nseg, w), jnp.float32),
               pltpu.VMEM((chunk,), jnp.int32),
               pltpu.VMEM((chunk, w), jnp.float32)]
    out_type = jax.ShapeDtypeStruct((_NC, nseg, w), jnp.float32)

    @functools.partial(pl.kernel, out_type=out_type, mesh=_MESH,
                       scratch_types=scratch,
                       compiler_params=pltpu.CompilerParams(
                           needs_layout_passes=False,
                           use_tc_tiling_on_sc=False))
    def k(rows_h, dst_h, zeros_h, out_h, acc_s, dst_v, rows_v):
        cid = lax.axis_index("c")
        sid = lax.axis_index("s")
        wid = sid * _NC + cid
        row0 = sid * rows_per_tile
        pltpu.sync_copy(zeros_h.at[pl.ds(row0, rows_per_tile)],
                        acc_s.at[pl.ds(row0, rows_per_tile)])
        plsc.subcore_barrier()
        nloc = (nch - wid + _NW - 1) // _NW

        def body(j, _):
            base = (wid + j * _NW) * chunk
            pltpu.sync_copy(dst_h.at[pl.ds(base, chunk)], dst_v)
            pltpu.sync_copy(rows_h.at[pl.ds(base, chunk)], rows_v)
            pltpu.sync_copy(rows_v, acc_s.at[dst_v], add=True)
            return 0

        lax.fori_loop(0, nloc, body, 0)
        plsc.subcore_barrier()
        pltpu.sync_copy(acc_s.at[pl.ds(row0, rows_per_tile)],
                        out_h.at[cid, pl.ds(row0, rows_per_tile)])

    return k(rows, dst, zeros)


def _score_body(qg, krtg, krog, et, eo):
    q = qg[...]
    et[...] = jnp.exp(jnp.sum(krtg[...] * q, axis=1, keepdims=True))
    eo[...] = jnp.exp(jnp.sum(krog[...] * q, axis=1, keepdims=True))


def _score_stage(qg, krtg, krog):
    be = 2000
    grid = (E // be,)
    return pl.pallas_call(
        _score_body, grid=grid,
        in_specs=[pl.BlockSpec((be, D), lambda i: (i, 0))] * 3,
        out_specs=[pl.BlockSpec((be, 1), lambda i: (i, 0))] * 2,
        out_shape=[jax.ShapeDtypeStruct((E, 1), jnp.float32)] * 2,
    )(qg, krtg, krog)


def _combine_body(pp, out):
    out[...] = jnp.sum(pp[...], axis=0)


def _combine_stage(partials):
    k, na, n = partials.shape
    return pl.pallas_call(
        _combine_body,
        out_shape=jax.ShapeDtypeStruct((na, n), jnp.float32),
    )(partials)


def _comb_rows_body(vrtg, vrog, et, eo, dtg, dog, comb):
    at = et[...] / (dtg[...] + 1e-16)
    ao = eo[...] / (dog[...] + 1e-16)
    comb[...] = at * vrtg[...] + ao * vrog[...]


def _comb_rows_stage(vrtg, vrog, et, eo, dtg, dog):
    be = 2000
    return pl.pallas_call(
        _comb_rows_body, grid=(E // be,),
        in_specs=[pl.BlockSpec((be, D), lambda i: (i, 0))] * 2 +
                 [pl.BlockSpec((be, 1), lambda i: (i, 0))] * 4,
        out_specs=pl.BlockSpec((be, D), lambda i: (i, 0)),
        out_shape=jax.ShapeDtypeStruct((E, D), jnp.float32),
    )(vrtg, vrog, et, eo, dtg.reshape(E, 1), dog.reshape(E, 1))


def _prep_body(ha2g, rag, ebg, rbg, eproj, rsum, geacc):
    ra = rag[...]
    rb = rbg[...]
    eproj[...] = ra[:, 30:31] + rb[:, 30:31]
    rsum[...] = ra + rb
    z = ha2g[...] + ebg[...]
    ez = jnp.exp(z)
    nm = jnp.sum(ez * z, axis=0, keepdims=True)
    dn = jnp.sum(ez, axis=0, keepdims=True)
    blk = jnp.concatenate([nm, dn], axis=0).reshape(1, 2, D)

    @pl.when(pl.program_id(0) == 0)
    def _():
        geacc[...] = blk

    @pl.when(pl.program_id(0) != 0)
    def _():
        geacc[...] += blk


def _prep_stage(ha2g, rag, ebg, rbg):
    be = 2000
    grid = (E // be,)
    in_specs = [pl.BlockSpec((be, D), lambda i: (i, 0)),
                pl.BlockSpec((be, 32), lambda i: (i, 0)),
                pl.BlockSpec((be, D), lambda i: (i, 0)),
                pl.BlockSpec((be, 32), lambda i: (i, 0))]
    out_specs = [pl.BlockSpec((be, 1), lambda i: (i, 0)),
                 pl.BlockSpec((be, 32), lambda i: (i, 0)),
                 pl.BlockSpec((1, 2, D), lambda i: (0, 0, 0))]
    out_shape = [jax.ShapeDtypeStruct((E, 1), jnp.float32),
                 jax.ShapeDtypeStruct((E, 32), jnp.float32),
                 jax.ShapeDtypeStruct((1, 2, D), jnp.float32)]
    return pl.pallas_call(
        _prep_body, grid=grid, in_specs=in_specs, out_specs=out_specs,
        out_shape=out_shape)(ha2g, rag, ebg, rbg)


def _node_body(aggp, x_node, awn, abn, glw, grw, scal, xl, xr, gnp):
    a = aggp[...]
    agg = a[0] + a[1]
    a_n = scal[0, 0]
    o = _gelu(agg) @ awn[...] + abn[...]
    hid = a_n * o + (1.0 - a_n) * x_node[...]
    xl[...] = hid @ glw[...] + scal[0, 1]
    xr[...] = hid @ grw[...] + scal[0, 2]
    mb = jnp.max(hid, axis=0, keepdims=True)
    @pl.when(pl.program_id(0) == 0)
    def _():
        ez = jnp.exp(hid - mb)
        gnp[...] = jnp.concatenate(
            [mb, jnp.sum(ez * hid, axis=0, keepdims=True),
             jnp.sum(ez, axis=0, keepdims=True)], axis=0)

    @pl.when(pl.program_id(0) != 0)
    def _():
        g = gnp[...]
        m_old = g[0:1]
        m_new = jnp.maximum(m_old, mb)
        ez = jnp.exp(hid - m_new)
        sc = jnp.exp(m_old - m_new)
        gnp[...] = jnp.concatenate(
            [m_new,
             g[1:2] * sc + jnp.sum(ez * hid, axis=0, keepdims=True),
             g[2:3] * sc + jnp.sum(ez, axis=0, keepdims=True)], axis=0)


def _node_stage(agg_partials, x_node, prm):
    bn = 2000
    scal = jnp.stack([jax.nn.sigmoid(prm['skip_node']),
                      prm['gat_l_b'][0], prm['gat_r_b'][0]]).reshape(1, 3)

    def rep(shape):
        return pl.BlockSpec(shape, lambda i: (0,) * len(shape))

    return pl.pallas_call(
        _node_body, grid=(N // bn,),
        in_specs=[pl.BlockSpec((2, bn, D), lambda i: (0, i, 0)),
                  pl.BlockSpec((bn, D), lambda i: (i, 0)),
                  rep((D, D)), rep((1, D)), rep((D, 1)), rep((D, 1)),
                  rep((1, 3))],
        out_specs=[pl.BlockSpec((bn, 1), lambda i: (i, 0)),
                   pl.BlockSpec((bn, 1), lambda i: (i, 0)),
                   pl.BlockSpec((3, D), lambda i: (0, 0))],
        out_shape=[jax.ShapeDtypeStruct((N, 1), jnp.float32),
                   jax.ShapeDtypeStruct((N, 1), jnp.float32),
                   jax.ShapeDtypeStruct((3, D), jnp.float32)],
    )(agg_partials, x_node, prm['a_w_node'], prm['a_b_node'].reshape(1, D),
      prm['gat_l_w'], prm['gat_r_w'], scal)


def _gat_edge_body(xls, xrd, ep, scal, es, esx):
    z = xls[...] + xrd[...] + ep[...]
    s = jnp.maximum(z, 0.2 * z) * scal[0, 0]
    e = jnp.exp(s)
    es[...] = e
    esx[...] = e * xls[...]


def _gat_edge_stage(xls, xrd, eproj, prm):
    be = 2000
    scal = prm['gat_att'].reshape(1, 1)
    return pl.pallas_call(
        _gat_edge_body, grid=(E // be,),
        in_specs=[pl.BlockSpec((be, 1), lambda i: (i, 0))] * 3 +
                 [pl.BlockSpec((1, 1), lambda i: (0, 0))],
        out_specs=[pl.BlockSpec((be, 1), lambda i: (i, 0))] * 2,
        out_shape=[jax.ShapeDtypeStruct((E, 1), jnp.float32)] * 2,
    )(xls.reshape(E, 1), xrd.reshape(E, 1), eproj, scal)


def _rootframe_body(gatp, cntr, xl, xr, gnp, gep, fw, fb, rwb, rb, scal,
                    root_preds, amax, frame, const32):
    g = jnp.sum(gatp[...], axis=0)
    cnt = cntr[...]
    att = scal[0, 0]
    bias = scal[0, 1]
    xlr = xl[...].reshape(1, N)
    xrr = xr[...].reshape(1, N)
    loop_eproj = g[0:1] / jnp.maximum(cnt, 1.0)
    z = xlr + xrr + loop_eproj
    s_self = jnp.maximum(z, 0.2 * z) * att
    es = jnp.exp(s_self)
    den = g[1:2] + es
    num = g[2:3] + es * xlr
    root = num / (den + 1e-16) + bias
    m = jnp.max(root, axis=1, keepdims=True)
    e = jnp.exp(root - m)
    lse = jnp.log(jnp.sum(e, axis=1, keepdims=True))
    root_preds[...] = root - m - lse
    idx = lax.broadcasted_iota(jnp.int32, (1, N), 1)
    amax[...] = jnp.min(jnp.where(root == m, idx, N), axis=1, keepdims=True)
    num2 = jnp.sum(gep[...], axis=0)
    ge = num2[0:1] / num2[1:2]
    gg = gnp[...]
    gn = gg[1:2] / gg[2:3]
    grep = jnp.concatenate([gn, ge], axis=1)
    f = grep @ fw[...] + fb[...]
    mf = jnp.max(f, axis=1, keepdims=True)
    lsef = jnp.log(jnp.sum(jnp.exp(f - mf), axis=1, keepdims=True))
    frame[...] = f - mf - lsef
    const32[...] = gn @ rwb[...] + rb[...]


def _rootframe_stage(gat_partials, cntr, xl, xr, gnp, ge_partials, prm,
                     rwb_pad, rb_pad):
    scal = jnp.stack([prm['gat_att'][0], prm['gat_bias'][0]]).reshape(1, 2)
    return pl.pallas_call(
        _rootframe_body,
        out_shape=[jax.ShapeDtypeStruct((1, N), jnp.float32),
                   jax.ShapeDtypeStruct((1, 1), jnp.int32),
                   jax.ShapeDtypeStruct((1, NF), jnp.float32),
                   jax.ShapeDtypeStruct((1, 32), jnp.float32)],
    )(gat_partials, cntr, xl, xr, gnp, ge_partials,
      prm['frame_w'], prm['frame_b'].reshape(1, NF), rwb_pad, rb_pad, scal)


def _role_body(rsum, src, amax, const32, out):
    r = rsum[...] + const32[...]
    keep = src[...] == amax[0, 0]
    r = jnp.where(keep, r, 0.0)
    lane = lax.broadcasted_iota(jnp.int32, r.shape, 1)
    valid = lane < NR
    rm = jnp.where(valid, r, _NEG)
    m = jnp.max(rm, axis=1, keepdims=True)
    e = jnp.where(valid, jnp.exp(r - m), 0.0)
    lse = jnp.log(jnp.sum(e, axis=1, keepdims=True))
    out[...] = r - m - lse


def _role_stage(rsum, edge_src, amax, const32):
    be = 2000
    grid = (E // be,)
    return pl.pallas_call(
        _role_body, grid=grid,
        in_specs=[pl.BlockSpec((be, 32), lambda i: (i, 0)),
                  pl.BlockSpec((be, 1), lambda i: (i, 0)),
                  pl.BlockSpec((1, 1), lambda i: (0, 0)),
                  pl.BlockSpec((1, 32), lambda i: (0, 0))],
        out_specs=pl.BlockSpec((be, 32), lambda i: (i, 0)),
        out_shape=jax.ShapeDtypeStruct((E, 32), jnp.float32),
    )(rsum, edge_src.reshape(E, 1), amax, const32)


def kernel(node_x, edge_x, edge_src, edge_dst, params):
    prm = params
    emb = prm['pred_emb']
    a_e = jax.nn.sigmoid(prm['skip_edge'])
    rwt2 = jnp.concatenate(
        [prm['role_w'][:D], prm['gat_e_w'],
         jnp.zeros((D, 32 - NR - 1), jnp.float32)], axis=1)
    rwb_pad = jnp.pad(prm['role_w'][D:], ((0, 0), (0, 32 - NR)))
    rb_pad = jnp.pad(prm['role_b'], (0, 32 - NR)).reshape(1, 32)

    q_tab, mcs, mce = _make_tables(emb, prm, a_e, rwt2)

    (x_node,) = _sc_gather_rows([emb], [node_x], chunk=80)

    (krtg, ha2g, vrtg, rag, krog, ebg, vrog, rbg, qg) = _sc_big_gather(
        mcs, mce, q_tab, node_x, edge_src, edge_dst, edge_x)

    et2, eo2 = _score_stage(qg, krtg, krog)
    den_partials = _sc_scatter_scalars(
        [et2.reshape(E), eo2.reshape(E)], edge_dst, N, count=True)
    dens = _combine_stage(den_partials)
    cntr = dens[2:3]

    eproj, rsum, ge_partials = _prep_stage(ha2g, rag, ebg, rbg)

    dtg, dog = _sc_gather_scalars([dens[0], dens[1]], [edge_dst, edge_dst])
    comb = _comb_rows_stage(vrtg, vrog, et2, eo2, dtg, dog)
    zeros_nd = jnp.zeros((N, D), jnp.float32)
    agg_partials = _sc_scatter_rows(comb, edge_dst, zeros_nd, N)

    xl, xr, gnp = _node_stage(agg_partials, x_node, prm)

    att16 = jnp.broadcast_to(prm['gat_att'][0], (_L,)).astype(jnp.float32)
    gat_partials = _sc_gat_fused(xl.reshape(N), xr.reshape(N),
                                 eproj.reshape(E), edge_src, edge_dst, att16)

    root_preds2, amax, frame2, const32 = _rootframe_stage(
        gat_partials, cntr, xl, xr, gnp, ge_partials, prm, rwb_pad, rb_pad)

    role32 = _role_stage(rsum, edge_src, amax, const32)

    root_preds = root_preds2.reshape(N)
    frame_preds = frame2.reshape(NF)
    role_preds = role32[:, :NR]
    return ((root_preds, frame_preds), role_preds)
